# Initial kernel scaffold; baseline (speedup 1.0000x reference)
import jax
import jax.numpy as jnp
from jax.experimental import pallas as pl


def kernel(x, edge_index, W1, att_src1, att_dst1, b1, W2, att_src2, att_dst2, b2):
    raise NotImplementedError("write your pallas kernel here")


# speedup vs baseline: 6.5657x; 6.5657x over previous
import functools

import jax
import jax.numpy as jnp
from jax import lax
from jax.experimental import pallas as pl
from jax.experimental.pallas import tpu as pltpu
from jax.experimental.pallas import tpu_sc as plsc

N = 10000
NP = 10240
E = 320000
F_IN = 128
HID = 64
HEADS = 8
G = 4
L = 16
TILES = 16
EPT = E // TILES
CH = 80
NCHUNK = EPT // CH
RPT = NP // TILES
UNROLL = 4


def _scA_body(asrcT, adstT, src, dst, bmax, zflat,
              eah, den,
              dtot0, dtot1,
              asrc0v, asrc1v, adst0v, adst1v,
              srcv, dstv, ea0, ea1, bmaxv):
    c = lax.axis_index("c")
    t = lax.axis_index("s")
    eo = t * EPT
    ro = t * RPT
    pltpu.sync_copy(bmax, bmaxv)

    def group_body(j, carry):
        g = 2 * c + j
        h0 = 2 * g
        pltpu.sync_copy(asrcT.at[pl.ds(h0 * NP, NP)], asrc0v)
        pltpu.sync_copy(asrcT.at[pl.ds((h0 + 1) * NP, NP)], asrc1v)
        pltpu.sync_copy(adstT.at[pl.ds(h0 * NP, NP)], adst0v)
        pltpu.sync_copy(adstT.at[pl.ds((h0 + 1) * NP, NP)], adst1v)
        pltpu.sync_copy(zflat.at[pl.ds(ro, RPT)], dtot0.at[pl.ds(ro, RPT)])
        pltpu.sync_copy(zflat.at[pl.ds(ro, RPT)], dtot1.at[pl.ds(ro, RPT)])
        plsc.subcore_barrier()
        b0 = plsc.load_gather(bmaxv, [jnp.full((L,), 0, jnp.int32) + h0])
        b1 = plsc.load_gather(bmaxv, [jnp.full((L,), 1, jnp.int32) + h0])

        def chunk_body(k, carry2):
            base = eo + k * CH
            pltpu.sync_copy(src.at[pl.ds(base, CH)], srcv)
            pltpu.sync_copy(dst.at[pl.ds(base, CH)], dstv)
            for s in range(CH // L):
                sv = srcv[pl.ds(s * L, L)]
                dv = dstv[pl.ds(s * L, L)]
                a0 = plsc.load_gather(asrc0v, [sv]) + plsc.load_gather(adst0v, [dv])
                a0 = jnp.where(a0 > 0, a0, a0 * jnp.float32(0.2))
                ea0[pl.ds(s * L, L)] = jnp.exp(a0 - b0)
                a1 = plsc.load_gather(asrc1v, [sv]) + plsc.load_gather(adst1v, [dv])
                a1 = jnp.where(a1 > 0, a1, a1 * jnp.float32(0.2))
                ea1[pl.ds(s * L, L)] = jnp.exp(a1 - b1)
            pltpu.sync_copy(ea0, dtot0.at[dstv], add=True)
            pltpu.sync_copy(ea1, dtot1.at[dstv], add=True)
            pltpu.sync_copy(ea0, eah.at[pl.ds(h0 * E + base, CH)])
            pltpu.sync_copy(ea1, eah.at[pl.ds((h0 + 1) * E + base, CH)])
            return carry2

        lax.fori_loop(0, NCHUNK, chunk_body, 0)
        plsc.subcore_barrier()
        pltpu.sync_copy(dtot0.at[pl.ds(ro, RPT)],
                        den.at[pl.ds(h0 * NP + ro, RPT)])
        pltpu.sync_copy(dtot1.at[pl.ds(ro, RPT)],
                        den.at[pl.ds((h0 + 1) * NP + ro, RPT)])
        plsc.subcore_barrier()
        return carry

    lax.fori_loop(0, 2, group_body, 0)


def _scB_body(hg, src, dst, eah, zrow,
              out1,
              acc,
              srcv, dstv, srcadj, rows, ea0, ea1, sem):
    c = lax.axis_index("c")
    t = lax.axis_index("s")
    eo = t * EPT
    ro = t * RPT

    def group_body(j, carry):
        g = 2 * c + j
        h0 = 2 * g
        goff = g * NP
        pltpu.sync_copy(zrow, acc.at[pl.ds(ro, RPT)])
        plsc.subcore_barrier()

        def chunk_body(k, carry2):
            base = eo + k * CH
            pltpu.sync_copy(src.at[pl.ds(base, CH)], srcv)
            pltpu.sync_copy(dst.at[pl.ds(base, CH)], dstv)
            for s in range(CH // L):
                sv = srcv[pl.ds(s * L, L)]
                srcadj[pl.ds(s * L, L)] = sv + goff
            gat = pltpu.async_copy(hg.at[srcadj], rows, sem)
            pltpu.sync_copy(eah.at[pl.ds(h0 * E + base, CH)], ea0)
            pltpu.sync_copy(eah.at[pl.ds((h0 + 1) * E + base, CH)], ea1)
            gat.wait()

            def scale_body(eb, carry3):
                for u in range(UNROLL):
                    e = eb * UNROLL + u
                    sp0 = plsc.load_gather(ea0, [jnp.full((L,), 0, jnp.int32) + e])
                    sp1 = plsc.load_gather(ea1, [jnp.full((L,), 0, jnp.int32) + e])
                    for r in range(8):
                        sp = sp0 if r < 4 else sp1
                        sl = pl.ds(r * L, L)
                        rows[e, sl] = rows[e, sl] * sp
                return carry3

            lax.fori_loop(0, CH // UNROLL, scale_body, 0)
            pltpu.sync_copy(rows, acc.at[dstv], add=True)
            return carry2

        lax.fori_loop(0, NCHUNK, chunk_body, 0)
        plsc.subcore_barrier()
        pltpu.sync_copy(acc.at[pl.ds(ro, RPT)],
                        out1.at[pl.ds(goff + ro, RPT)])
        plsc.subcore_barrier()
        return carry

    lax.fori_loop(0, 2, group_body, 0)


@jax.jit
def _sc1(hg, asrcT, adstT, src, dst, bmax, zrow, zflat):
    mesh = plsc.VectorSubcoreMesh(core_axis_name="c", subcore_axis_name="s")
    fA = pl.kernel(
        _scA_body,
        out_type=[
            jax.ShapeDtypeStruct((HEADS * E,), jnp.float32),
            jax.ShapeDtypeStruct((HEADS * NP,), jnp.float32),
        ],
        mesh=mesh,
        compiler_params=pltpu.CompilerParams(needs_layout_passes=False),
        scratch_types=[
            pltpu.VMEM_SHARED((NP,), jnp.float32),
            pltpu.VMEM_SHARED((NP,), jnp.float32),
            pltpu.VMEM((NP,), jnp.float32),
            pltpu.VMEM((NP,), jnp.float32),
            pltpu.VMEM((NP,), jnp.float32),
            pltpu.VMEM((NP,), jnp.float32),
            pltpu.VMEM((CH,), jnp.int32),
            pltpu.VMEM((CH,), jnp.int32),
            pltpu.VMEM((CH,), jnp.float32),
            pltpu.VMEM((CH,), jnp.float32),
            pltpu.VMEM((L,), jnp.float32),
        ],
    )
    eah, den = fA(asrcT, adstT, src, dst, bmax, zflat)
    fB = pl.kernel(
        _scB_body,
        out_type=[
            jax.ShapeDtypeStruct((G * NP, 128), jnp.float32),
        ],
        mesh=mesh,
        compiler_params=pltpu.CompilerParams(needs_layout_passes=False),
        scratch_types=[
            pltpu.VMEM_SHARED((NP, 128), jnp.float32),
            pltpu.VMEM((CH,), jnp.int32),
            pltpu.VMEM((CH,), jnp.int32),
            pltpu.VMEM((CH,), jnp.int32),
            pltpu.VMEM((CH, 128), jnp.float32),
            pltpu.VMEM((CH,), jnp.float32),
            pltpu.VMEM((CH,), jnp.float32),
            pltpu.SemaphoreType.DMA,
        ],
    )
    (out1f,) = fB(hg, src, dst, eah, zrow)
    return out1f, den


def kernel(x, edge_index, W1, att_src1, att_dst1, b1, W2, att_src2, att_dst2, b2):
    f32 = jnp.float32
    src = edge_index[0]
    dst = edge_index[1]
    xp = jnp.pad(x, ((0, NP - N), (0, 0)))

    h = xp @ W1
    hr = h.reshape(NP, HEADS, HID)
    asrc = (hr * att_src1[None]).sum(-1)
    adst = (hr * att_dst1[None]).sum(-1)
    gm = asrc.max(0) + adst.max(0)
    bmax = jnp.where(gm > 0, gm, gm * f32(0.2))
    bmax16 = jnp.pad(bmax, (0, 16 - HEADS))
    hg = h.reshape(NP, G, 128).transpose(1, 0, 2).reshape(G * NP, 128)
    asrcT = asrc.T.reshape(-1)
    adstT = adst.T.reshape(-1)
    zrow = jnp.zeros((RPT, 128), f32)
    zflat = jnp.zeros((NP,), f32)

    out1f, denf = _sc1(hg, asrcT, adstT, src, dst, bmax16, zrow, zflat)

    out1 = out1f.reshape(G, NP, 2, HID)
    den = denf.reshape(HEADS, NP).T.reshape(NP, G, 2)
    den = den.transpose(1, 0, 2)[:, :, :, None]
    out1 = out1 / (den + f32(1e-16))
    h1 = out1.reshape(G, NP, 128).transpose(1, 0, 2).reshape(NP, HEADS * HID)
    h1 = jax.nn.elu(h1 + b1)

    h2 = h1 @ W2
    as2 = (h2 * att_src2[0][None]).sum(-1)
    ad2 = (h2 * att_dst2[0][None]).sum(-1)
    gm2 = as2.max() + ad2.max()
    bm2 = jnp.where(gm2 > 0, gm2, gm2 * f32(0.2))
    al2 = as2[src] + ad2[dst]
    al2 = jnp.where(al2 > 0, al2, al2 * f32(0.2))
    ea2 = jnp.exp(al2 - bm2)
    d2 = jax.ops.segment_sum(ea2, dst, num_segments=NP)
    o2 = jax.ops.segment_sum(ea2[:, None] * h2[src], dst, num_segments=NP)
    out = o2 / (d2[:, None] + f32(1e-16)) + b2
    return out[:N]


# speedup vs baseline: 18.8207x; 2.8665x over previous
import functools

import jax
import jax.numpy as jnp
from jax import lax
from jax.experimental import pallas as pl
from jax.experimental.pallas import tpu as pltpu
from jax.experimental.pallas import tpu_sc as plsc

N = 10000
NP = 10240
E = 320000
F_IN = 128
HID = 64
HEADS = 8
G = 4
L = 16
TILES = 16
EPT = E // TILES
CH = 80
NCHUNK = EPT // CH
RPT = NP // TILES
UNROLL = 4
BN1 = 512
BN2 = 256


def _tc1_body(x_ref, W1_ref, As_ref, Ad_ref, h_ref, as_ref, ad_ref):
    h = jnp.dot(x_ref[...], W1_ref[...], preferred_element_type=jnp.float32)
    h_ref[...] = h
    as_ref[...] = jnp.dot(h, As_ref[...], preferred_element_type=jnp.float32)
    ad_ref[...] = jnp.dot(h, Ad_ref[...], preferred_element_type=jnp.float32)


def _tc1(xp, W1, As, Ad):
    return pl.pallas_call(
        _tc1_body,
        grid=(NP // BN1,),
        in_specs=[
            pl.BlockSpec((BN1, F_IN), lambda i: (i, 0)),
            pl.BlockSpec((F_IN, HEADS * HID), lambda i: (0, 0)),
            pl.BlockSpec((HEADS * HID, HEADS), lambda i: (0, 0)),
            pl.BlockSpec((HEADS * HID, HEADS), lambda i: (0, 0)),
        ],
        out_specs=[
            pl.BlockSpec((BN1, HEADS * HID), lambda i: (i, 0)),
            pl.BlockSpec((BN1, HEADS), lambda i: (i, 0)),
            pl.BlockSpec((BN1, HEADS), lambda i: (i, 0)),
        ],
        out_shape=[
            jax.ShapeDtypeStruct((NP, HEADS * HID), jnp.float32),
            jax.ShapeDtypeStruct((NP, HEADS), jnp.float32),
            jax.ShapeDtypeStruct((NP, HEADS), jnp.float32),
        ],
    )(xp, W1, As, Ad)


def _tc2_body(or_ref, dent_ref, b1_ref, W2_ref, a2m_ref, h2_ref, asd_ref):
    f32 = jnp.float32
    parts = []
    for g in range(G):
        d0 = jnp.broadcast_to(dent_ref[g, :, 0:1], (BN2, HID))
        d1 = jnp.broadcast_to(dent_ref[g, :, 1:2], (BN2, HID))
        dcat = jnp.concatenate([d0, d1], axis=1)
        parts.append(or_ref[g] / (dcat + f32(1e-16)))
    h1 = jnp.concatenate(parts, axis=1) + b1_ref[...]
    h1 = jnp.where(h1 > 0, h1, jnp.exp(jnp.minimum(h1, 0.0)) - f32(1.0))
    h2 = jnp.dot(h1, W2_ref[...], preferred_element_type=f32)
    h2_ref[...] = h2
    asd_ref[...] = jnp.dot(h2, a2m_ref[...], preferred_element_type=f32)


def _tc2(or3, dent, b1r, W2, a2m):
    return pl.pallas_call(
        _tc2_body,
        grid=(NP // BN2,),
        in_specs=[
            pl.BlockSpec((G, BN2, 2 * HID), lambda i: (0, i, 0)),
            pl.BlockSpec((G, BN2, 2), lambda i: (0, i, 0)),
            pl.BlockSpec((1, HEADS * HID), lambda i: (0, 0)),
            pl.BlockSpec((HEADS * HID, 2), lambda i: (0, 0)),
            pl.BlockSpec((2, 2), lambda i: (0, 0)),
        ],
        out_specs=[
            pl.BlockSpec((BN2, 2), lambda i: (i, 0)),
            pl.BlockSpec((BN2, 2), lambda i: (i, 0)),
        ],
        out_shape=[
            jax.ShapeDtypeStruct((NP, 2), jnp.float32),
            jax.ShapeDtypeStruct((NP, 2), jnp.float32),
        ],
    )(or3, dent, b1r, W2, a2m)


def _sc2_body(as2T, ad2T, h2c, src, dst, bm2, b2b, zflat,
              out2,
              o0s, o1s, d2s,
              as2v, ad2v, h0v, h1v, srcv, dstv, eab, v0b, v1b,
              bm2v, b2v, s0, s1, s2, ob):
    c = lax.axis_index("c")
    t = lax.axis_index("s")
    eo = t * EPT
    ro = t * RPT

    @pl.when(c == 0)
    def _():
        pltpu.sync_copy(as2T, as2v)
        pltpu.sync_copy(ad2T, ad2v)
        pltpu.sync_copy(h2c.at[pl.ds(0, NP)], h0v)
        pltpu.sync_copy(h2c.at[pl.ds(NP, NP)], h1v)
        pltpu.sync_copy(bm2, bm2v)
        pltpu.sync_copy(b2b, b2v)
        pltpu.sync_copy(zflat.at[pl.ds(ro, RPT)], o0s.at[pl.ds(ro, RPT)])
        pltpu.sync_copy(zflat.at[pl.ds(ro, RPT)], o1s.at[pl.ds(ro, RPT)])
        pltpu.sync_copy(zflat.at[pl.ds(ro, RPT)], d2s.at[pl.ds(ro, RPT)])
        plsc.subcore_barrier()
        bmv = bm2v[...]

        def chunk_body(k, carry):
            base = eo + k * CH
            pltpu.sync_copy(src.at[pl.ds(base, CH)], srcv)
            pltpu.sync_copy(dst.at[pl.ds(base, CH)], dstv)
            for s in range(CH // L):
                sv = srcv[pl.ds(s * L, L)]
                dv = dstv[pl.ds(s * L, L)]
                a = plsc.load_gather(as2v, [sv]) + plsc.load_gather(ad2v, [dv])
                a = jnp.where(a > 0, a, a * jnp.float32(0.2))
                ea = jnp.exp(a - bmv)
                eab[pl.ds(s * L, L)] = ea
                v0b[pl.ds(s * L, L)] = ea * plsc.load_gather(h0v, [sv])
                v1b[pl.ds(s * L, L)] = ea * plsc.load_gather(h1v, [sv])
            pltpu.sync_copy(eab, d2s.at[dstv], add=True)
            pltpu.sync_copy(v0b, o0s.at[dstv], add=True)
            pltpu.sync_copy(v1b, o1s.at[dstv], add=True)
            return carry

        lax.fori_loop(0, NCHUNK, chunk_body, 0)
        plsc.subcore_barrier()
        pltpu.sync_copy(o0s.at[pl.ds(ro, RPT)], s0)
        pltpu.sync_copy(o1s.at[pl.ds(ro, RPT)], s1)
        pltpu.sync_copy(d2s.at[pl.ds(ro, RPT)], s2)
        bb0 = b2v[pl.ds(0, L)]
        bb1 = b2v[pl.ds(L, L)]

        def out_body(i, carry):
            sl = pl.ds(i * L, L)
            d = s2[sl] + jnp.float32(1e-16)
            ob[sl] = s0[sl] / d + bb0
            s0[sl] = s1[sl] / d + bb1
            return carry

        lax.fori_loop(0, RPT // L, out_body, 0)
        pltpu.sync_copy(ob, out2.at[pl.ds(ro, RPT)])
        pltpu.sync_copy(s0, out2.at[pl.ds(NP + ro, RPT)])


def _sc2(as2T, ad2T, h2c, src, dst, bm2, b2b, zflat):
    mesh = plsc.VectorSubcoreMesh(core_axis_name="c", subcore_axis_name="s")
    f = pl.kernel(
        _sc2_body,
        out_type=[jax.ShapeDtypeStruct((2 * NP,), jnp.float32)],
        mesh=mesh,
        compiler_params=pltpu.CompilerParams(needs_layout_passes=False),
        scratch_types=[
            pltpu.VMEM_SHARED((NP,), jnp.float32),
            pltpu.VMEM_SHARED((NP,), jnp.float32),
            pltpu.VMEM_SHARED((NP,), jnp.float32),
            pltpu.VMEM((NP,), jnp.float32),
            pltpu.VMEM((NP,), jnp.float32),
            pltpu.VMEM((NP,), jnp.float32),
            pltpu.VMEM((NP,), jnp.float32),
            pltpu.VMEM((CH,), jnp.int32),
            pltpu.VMEM((CH,), jnp.int32),
            pltpu.VMEM((CH,), jnp.float32),
            pltpu.VMEM((CH,), jnp.float32),
            pltpu.VMEM((CH,), jnp.float32),
            pltpu.VMEM((L,), jnp.float32),
            pltpu.VMEM((2 * L,), jnp.float32),
            pltpu.VMEM((RPT,), jnp.float32),
            pltpu.VMEM((RPT,), jnp.float32),
            pltpu.VMEM((RPT,), jnp.float32),
            pltpu.VMEM((RPT,), jnp.float32),
        ],
    )
    return f(as2T, ad2T, h2c, src, dst, bm2, b2b, zflat)


def _scA_body(asrcT, adstT, src, dst, bmax, zflat,
              eah, den,
              dtot0, dtot1,
              asrc0v, asrc1v, adst0v, adst1v,
              srcv, dstv, ea0, ea1, bmaxv):
    c = lax.axis_index("c")
    t = lax.axis_index("s")
    eo = t * EPT
    ro = t * RPT
    pltpu.sync_copy(bmax, bmaxv)

    def group_body(j, carry):
        g = 2 * c + j
        h0 = 2 * g
        pltpu.sync_copy(asrcT.at[pl.ds(h0 * NP, NP)], asrc0v)
        pltpu.sync_copy(asrcT.at[pl.ds((h0 + 1) * NP, NP)], asrc1v)
        pltpu.sync_copy(adstT.at[pl.ds(h0 * NP, NP)], adst0v)
        pltpu.sync_copy(adstT.at[pl.ds((h0 + 1) * NP, NP)], adst1v)
        pltpu.sync_copy(zflat.at[pl.ds(ro, RPT)], dtot0.at[pl.ds(ro, RPT)])
        pltpu.sync_copy(zflat.at[pl.ds(ro, RPT)], dtot1.at[pl.ds(ro, RPT)])
        plsc.subcore_barrier()
        b0 = plsc.load_gather(bmaxv, [jnp.full((L,), 0, jnp.int32) + h0])
        b1 = plsc.load_gather(bmaxv, [jnp.full((L,), 1, jnp.int32) + h0])

        def chunk_body(k, carry2):
            base = eo + k * CH
            pltpu.sync_copy(src.at[pl.ds(base, CH)], srcv)
            pltpu.sync_copy(dst.at[pl.ds(base, CH)], dstv)
            for s in range(CH // L):
                sv = srcv[pl.ds(s * L, L)]
                dv = dstv[pl.ds(s * L, L)]
                a0 = plsc.load_gather(asrc0v, [sv]) + plsc.load_gather(adst0v, [dv])
                a0 = jnp.where(a0 > 0, a0, a0 * jnp.float32(0.2))
                ea0[pl.ds(s * L, L)] = jnp.exp(a0 - b0)
                a1 = plsc.load_gather(asrc1v, [sv]) + plsc.load_gather(adst1v, [dv])
                a1 = jnp.where(a1 > 0, a1, a1 * jnp.float32(0.2))
                ea1[pl.ds(s * L, L)] = jnp.exp(a1 - b1)
            pltpu.sync_copy(ea0, dtot0.at[dstv], add=True)
            pltpu.sync_copy(ea1, dtot1.at[dstv], add=True)
            pltpu.sync_copy(ea0, eah.at[pl.ds(h0 * E + base, CH)])
            pltpu.sync_copy(ea1, eah.at[pl.ds((h0 + 1) * E + base, CH)])
            return carry2

        lax.fori_loop(0, NCHUNK, chunk_body, 0)
        plsc.subcore_barrier()
        pltpu.sync_copy(dtot0.at[pl.ds(ro, RPT)],
                        den.at[pl.ds(h0 * NP + ro, RPT)])
        pltpu.sync_copy(dtot1.at[pl.ds(ro, RPT)],
                        den.at[pl.ds((h0 + 1) * NP + ro, RPT)])
        plsc.subcore_barrier()
        return carry

    lax.fori_loop(0, 2, group_body, 0)


def _scB_body(hg, src, dst, eah, zrow,
              out1,
              acc,
              srcv, dstv, srcadj, rows, ea0, ea1, sem):
    c = lax.axis_index("c")
    t = lax.axis_index("s")
    eo = t * EPT
    ro = t * RPT

    def group_body(j, carry):
        g = 2 * c + j
        h0 = 2 * g
        goff = g * NP
        pltpu.sync_copy(zrow, acc.at[pl.ds(ro, RPT)])
        plsc.subcore_barrier()

        def chunk_body(k, carry2):
            base = eo + k * CH
            pltpu.sync_copy(src.at[pl.ds(base, CH)], srcv)
            pltpu.sync_copy(dst.at[pl.ds(base, CH)], dstv)
            for s in range(CH // L):
                sv = srcv[pl.ds(s * L, L)]
                srcadj[pl.ds(s * L, L)] = sv + goff
            gat = pltpu.async_copy(hg.at[srcadj], rows, sem)
            pltpu.sync_copy(eah.at[pl.ds(h0 * E + base, CH)], ea0)
            pltpu.sync_copy(eah.at[pl.ds((h0 + 1) * E + base, CH)], ea1)
            gat.wait()

            def scale_body(eb, carry3):
                for u in range(UNROLL):
                    e = eb * UNROLL + u
                    sp0 = plsc.load_gather(ea0, [jnp.full((L,), 0, jnp.int32) + e])
                    sp1 = plsc.load_gather(ea1, [jnp.full((L,), 0, jnp.int32) + e])
                    for r in range(8):
                        sp = sp0 if r < 4 else sp1
                        sl = pl.ds(r * L, L)
                        rows[e, sl] = rows[e, sl] * sp
                return carry3

            lax.fori_loop(0, CH // UNROLL, scale_body, 0)
            pltpu.sync_copy(rows, acc.at[dstv], add=True)
            return carry2

        lax.fori_loop(0, NCHUNK, chunk_body, 0)
        plsc.subcore_barrier()
        pltpu.sync_copy(acc.at[pl.ds(ro, RPT)],
                        out1.at[pl.ds(goff + ro, RPT)])
        plsc.subcore_barrier()
        return carry

    lax.fori_loop(0, 2, group_body, 0)


@jax.jit
def _sc1(hg, asrcT, adstT, src, dst, bmax, zrow, zflat):
    mesh = plsc.VectorSubcoreMesh(core_axis_name="c", subcore_axis_name="s")
    fA = pl.kernel(
        _scA_body,
        out_type=[
            jax.ShapeDtypeStruct((HEADS * E,), jnp.float32),
            jax.ShapeDtypeStruct((HEADS * NP,), jnp.float32),
        ],
        mesh=mesh,
        compiler_params=pltpu.CompilerParams(needs_layout_passes=False),
        scratch_types=[
            pltpu.VMEM_SHARED((NP,), jnp.float32),
            pltpu.VMEM_SHARED((NP,), jnp.float32),
            pltpu.VMEM((NP,), jnp.float32),
            pltpu.VMEM((NP,), jnp.float32),
            pltpu.VMEM((NP,), jnp.float32),
            pltpu.VMEM((NP,), jnp.float32),
            pltpu.VMEM((CH,), jnp.int32),
            pltpu.VMEM((CH,), jnp.int32),
            pltpu.VMEM((CH,), jnp.float32),
            pltpu.VMEM((CH,), jnp.float32),
            pltpu.VMEM((L,), jnp.float32),
        ],
    )
    eah, den = fA(asrcT, adstT, src, dst, bmax, zflat)
    fB = pl.kernel(
        _scB_body,
        out_type=[
            jax.ShapeDtypeStruct((G * NP, 128), jnp.float32),
        ],
        mesh=mesh,
        compiler_params=pltpu.CompilerParams(needs_layout_passes=False),
        scratch_types=[
            pltpu.VMEM_SHARED((NP, 128), jnp.float32),
            pltpu.VMEM((CH,), jnp.int32),
            pltpu.VMEM((CH,), jnp.int32),
            pltpu.VMEM((CH,), jnp.int32),
            pltpu.VMEM((CH, 128), jnp.float32),
            pltpu.VMEM((CH,), jnp.float32),
            pltpu.VMEM((CH,), jnp.float32),
            pltpu.SemaphoreType.DMA,
        ],
    )
    (out1f,) = fB(hg, src, dst, eah, zrow)
    return out1f, den


def kernel(x, edge_index, W1, att_src1, att_dst1, b1, W2, att_src2, att_dst2, b2):
    f32 = jnp.float32
    src = edge_index[0]
    dst = edge_index[1]
    xp = jnp.pad(x, ((0, NP - N), (0, 0)))

    As = jnp.zeros((HEADS * HID, HEADS), f32)
    Ad = jnp.zeros((HEADS * HID, HEADS), f32)
    for hh in range(HEADS):
        As = As.at[hh * HID:(hh + 1) * HID, hh].set(att_src1[hh])
        Ad = Ad.at[hh * HID:(hh + 1) * HID, hh].set(att_dst1[hh])

    h, asrc, adst = _tc1(xp, W1, As, Ad)

    gm = asrc.max(0) + adst.max(0)
    bmax = jnp.where(gm > 0, gm, gm * f32(0.2))
    bmax16 = jnp.pad(bmax, (0, 16 - HEADS))
    hg = h.reshape(NP, G, 128).transpose(1, 0, 2).reshape(G * NP, 128)
    asrcT = asrc.T.reshape(-1)
    adstT = adst.T.reshape(-1)
    zrow = jnp.zeros((RPT, 128), f32)
    zflat = jnp.zeros((NP,), f32)

    out1f, denf = _sc1(hg, asrcT, adstT, src, dst, bmax16, zrow, zflat)

    or3 = out1f.reshape(G, NP, 128)
    dent = denf.reshape(G, 2, NP).transpose(0, 2, 1)
    a2m = jnp.stack([att_src2[0], att_dst2[0]], axis=1)
    h2, asd = _tc2(or3, dent, b1.reshape(1, HEADS * HID), W2, a2m)

    as2 = asd[:, 0]
    ad2 = asd[:, 1]
    gm2 = as2.max() + ad2.max()
    bm2 = jnp.where(gm2 > 0, gm2, gm2 * f32(0.2))
    bm2v = jnp.broadcast_to(bm2[None], (L,))
    b2b = jnp.broadcast_to(b2[:, None], (2, L)).reshape(-1)
    h2c = h2.T.reshape(-1)

    (out2,) = _sc2(as2, ad2, h2c, src, dst, bm2v, b2b, zflat)
    return out2.reshape(2, NP).T[:N]


# speedup vs baseline: 20.3679x; 1.0822x over previous
import functools

import jax
import jax.numpy as jnp
from jax import lax
from jax.experimental import pallas as pl
from jax.experimental.pallas import tpu as pltpu
from jax.experimental.pallas import tpu_sc as plsc

N = 10000
NP = 10240
E = 320000
F_IN = 128
HID = 64
HEADS = 8
G = 4
L = 16
TILES = 16
EPT = E // TILES
CH = 80
NCHUNK = EPT // CH
RPT = NP // TILES
UNROLL = 4
BN1 = 512
BN2 = 256


def _tc1_body(x_ref, W1_ref, As_ref, Ad_ref, h_ref, as_ref, ad_ref):
    h = jnp.dot(x_ref[...], W1_ref[...], preferred_element_type=jnp.float32)
    h_ref[...] = h
    as_ref[...] = jnp.dot(h, As_ref[...], preferred_element_type=jnp.float32)
    ad_ref[...] = jnp.dot(h, Ad_ref[...], preferred_element_type=jnp.float32)


def _tc1(xp, W1, As, Ad):
    return pl.pallas_call(
        _tc1_body,
        grid=(NP // BN1,),
        in_specs=[
            pl.BlockSpec((BN1, F_IN), lambda i: (i, 0)),
            pl.BlockSpec((F_IN, HEADS * HID), lambda i: (0, 0)),
            pl.BlockSpec((HEADS * HID, HEADS), lambda i: (0, 0)),
            pl.BlockSpec((HEADS * HID, HEADS), lambda i: (0, 0)),
        ],
        out_specs=[
            pl.BlockSpec((BN1, HEADS * HID), lambda i: (i, 0)),
            pl.BlockSpec((BN1, HEADS), lambda i: (i, 0)),
            pl.BlockSpec((BN1, HEADS), lambda i: (i, 0)),
        ],
        out_shape=[
            jax.ShapeDtypeStruct((NP, HEADS * HID), jnp.float32),
            jax.ShapeDtypeStruct((NP, HEADS), jnp.float32),
            jax.ShapeDtypeStruct((NP, HEADS), jnp.float32),
        ],
    )(xp, W1, As, Ad)


def _tc2_body(or_ref, dent_ref, b1_ref, W2_ref, a2m_ref, h2_ref, asd_ref):
    f32 = jnp.float32
    parts = []
    for g in range(G):
        d0 = jnp.broadcast_to(dent_ref[g, :, 0:1], (BN2, HID))
        d1 = jnp.broadcast_to(dent_ref[g, :, 1:2], (BN2, HID))
        dcat = jnp.concatenate([d0, d1], axis=1)
        parts.append(or_ref[g] / (dcat + f32(1e-16)))
    h1 = jnp.concatenate(parts, axis=1) + b1_ref[...]
    h1 = jnp.where(h1 > 0, h1, jnp.exp(jnp.minimum(h1, 0.0)) - f32(1.0))
    h2 = jnp.dot(h1, W2_ref[...], preferred_element_type=f32)
    h2_ref[...] = h2
    asd_ref[...] = jnp.dot(h2, a2m_ref[...], preferred_element_type=f32)


def _tc2(or3, dent, b1r, W2, a2m):
    return pl.pallas_call(
        _tc2_body,
        grid=(NP // BN2,),
        in_specs=[
            pl.BlockSpec((G, BN2, 2 * HID), lambda i: (0, i, 0)),
            pl.BlockSpec((G, BN2, 2), lambda i: (0, i, 0)),
            pl.BlockSpec((1, HEADS * HID), lambda i: (0, 0)),
            pl.BlockSpec((HEADS * HID, 2), lambda i: (0, 0)),
            pl.BlockSpec((2, 2), lambda i: (0, 0)),
        ],
        out_specs=[
            pl.BlockSpec((BN2, 2), lambda i: (i, 0)),
            pl.BlockSpec((BN2, 2), lambda i: (i, 0)),
        ],
        out_shape=[
            jax.ShapeDtypeStruct((NP, 2), jnp.float32),
            jax.ShapeDtypeStruct((NP, 2), jnp.float32),
        ],
    )(or3, dent, b1r, W2, a2m)


def _sc2_body(as2T, ad2T, h2c, src, dst, bm2, b2b, zflat,
              out2,
              o0s, o1s, d2s,
              as2v, ad2v, h0v, h1v, srcv, dstv, eab, v0b, v1b,
              bm2v, b2v, s0, s1, s2, ob):
    c = lax.axis_index("c")
    t = lax.axis_index("s")
    eo = t * EPT
    ro = t * RPT

    @pl.when(c == 0)
    def _():
        pltpu.sync_copy(as2T, as2v)
        pltpu.sync_copy(ad2T, ad2v)
        pltpu.sync_copy(h2c.at[pl.ds(0, NP)], h0v)
        pltpu.sync_copy(h2c.at[pl.ds(NP, NP)], h1v)
        pltpu.sync_copy(bm2, bm2v)
        pltpu.sync_copy(b2b, b2v)
        pltpu.sync_copy(zflat.at[pl.ds(ro, RPT)], o0s.at[pl.ds(ro, RPT)])
        pltpu.sync_copy(zflat.at[pl.ds(ro, RPT)], o1s.at[pl.ds(ro, RPT)])
        pltpu.sync_copy(zflat.at[pl.ds(ro, RPT)], d2s.at[pl.ds(ro, RPT)])
        plsc.subcore_barrier()
        bmv = bm2v[...]

        def chunk_body(k, carry):
            base = eo + k * CH
            pltpu.sync_copy(src.at[pl.ds(base, CH)], srcv)
            pltpu.sync_copy(dst.at[pl.ds(base, CH)], dstv)
            for s in range(CH // L):
                sv = srcv[pl.ds(s * L, L)]
                dv = dstv[pl.ds(s * L, L)]
                a = plsc.load_gather(as2v, [sv]) + plsc.load_gather(ad2v, [dv])
                a = jnp.where(a > 0, a, a * jnp.float32(0.2))
                ea = jnp.exp(a - bmv)
                eab[pl.ds(s * L, L)] = ea
                v0b[pl.ds(s * L, L)] = ea * plsc.load_gather(h0v, [sv])
                v1b[pl.ds(s * L, L)] = ea * plsc.load_gather(h1v, [sv])
            pltpu.sync_copy(eab, d2s.at[dstv], add=True)
            pltpu.sync_copy(v0b, o0s.at[dstv], add=True)
            pltpu.sync_copy(v1b, o1s.at[dstv], add=True)
            return carry

        lax.fori_loop(0, NCHUNK, chunk_body, 0)
        plsc.subcore_barrier()
        pltpu.sync_copy(o0s.at[pl.ds(ro, RPT)], s0)
        pltpu.sync_copy(o1s.at[pl.ds(ro, RPT)], s1)
        pltpu.sync_copy(d2s.at[pl.ds(ro, RPT)], s2)
        bb0 = b2v[pl.ds(0, L)]
        bb1 = b2v[pl.ds(L, L)]

        def out_body(i, carry):
            sl = pl.ds(i * L, L)
            d = s2[sl] + jnp.float32(1e-16)
            ob[sl] = s0[sl] / d + bb0
            s0[sl] = s1[sl] / d + bb1
            return carry

        lax.fori_loop(0, RPT // L, out_body, 0)
        pltpu.sync_copy(ob, out2.at[pl.ds(ro, RPT)])
        pltpu.sync_copy(s0, out2.at[pl.ds(NP + ro, RPT)])


def _sc2(as2T, ad2T, h2c, src, dst, bm2, b2b, zflat):
    mesh = plsc.VectorSubcoreMesh(core_axis_name="c", subcore_axis_name="s")
    f = pl.kernel(
        _sc2_body,
        out_type=[jax.ShapeDtypeStruct((2 * NP,), jnp.float32)],
        mesh=mesh,
        compiler_params=pltpu.CompilerParams(needs_layout_passes=False),
        scratch_types=[
            pltpu.VMEM_SHARED((NP,), jnp.float32),
            pltpu.VMEM_SHARED((NP,), jnp.float32),
            pltpu.VMEM_SHARED((NP,), jnp.float32),
            pltpu.VMEM((NP,), jnp.float32),
            pltpu.VMEM((NP,), jnp.float32),
            pltpu.VMEM((NP,), jnp.float32),
            pltpu.VMEM((NP,), jnp.float32),
            pltpu.VMEM((CH,), jnp.int32),
            pltpu.VMEM((CH,), jnp.int32),
            pltpu.VMEM((CH,), jnp.float32),
            pltpu.VMEM((CH,), jnp.float32),
            pltpu.VMEM((CH,), jnp.float32),
            pltpu.VMEM((L,), jnp.float32),
            pltpu.VMEM((2 * L,), jnp.float32),
            pltpu.VMEM((RPT,), jnp.float32),
            pltpu.VMEM((RPT,), jnp.float32),
            pltpu.VMEM((RPT,), jnp.float32),
            pltpu.VMEM((RPT,), jnp.float32),
        ],
    )
    return f(as2T, ad2T, h2c, src, dst, bm2, b2b, zflat)


def _scA_body(asrcT, adstT, src, dst, bmax, zflat,
              eah, den,
              dtot0, dtot1,
              asrc0v, asrc1v, adst0v, adst1v,
              srcvA, dstvA, ea0A, ea1A,
              srcvB, dstvB, ea0B, ea1B,
              bmaxv, sem1, sem2):
    c = lax.axis_index("c")
    t = lax.axis_index("s")
    eo = t * EPT
    ro = t * RPT
    pltpu.sync_copy(bmax, bmaxv)

    def group_body(j, carry):
        g = 2 * c + j
        h0 = 2 * g
        pltpu.sync_copy(asrcT.at[pl.ds(h0 * NP, NP)], asrc0v)
        pltpu.sync_copy(asrcT.at[pl.ds((h0 + 1) * NP, NP)], asrc1v)
        pltpu.sync_copy(adstT.at[pl.ds(h0 * NP, NP)], adst0v)
        pltpu.sync_copy(adstT.at[pl.ds((h0 + 1) * NP, NP)], adst1v)
        pltpu.sync_copy(zflat.at[pl.ds(ro, RPT)], dtot0.at[pl.ds(ro, RPT)])
        pltpu.sync_copy(zflat.at[pl.ds(ro, RPT)], dtot1.at[pl.ds(ro, RPT)])
        plsc.subcore_barrier()
        b0 = plsc.load_gather(bmaxv, [jnp.full((L,), 0, jnp.int32) + h0])
        b1 = plsc.load_gather(bmaxv, [jnp.full((L,), 1, jnp.int32) + h0])

        def compute_ea(srcv, dstv, ea0, ea1):
            for s in range(CH // L):
                sv = srcv[pl.ds(s * L, L)]
                dv = dstv[pl.ds(s * L, L)]
                a0 = plsc.load_gather(asrc0v, [sv]) + plsc.load_gather(adst0v, [dv])
                a0 = jnp.where(a0 > 0, a0, a0 * jnp.float32(0.2))
                ea0[pl.ds(s * L, L)] = jnp.exp(a0 - b0)
                a1 = plsc.load_gather(asrc1v, [sv]) + plsc.load_gather(adst1v, [dv])
                a1 = jnp.where(a1 > 0, a1, a1 * jnp.float32(0.2))
                ea1[pl.ds(s * L, L)] = jnp.exp(a1 - b1)

        def fire(base, dstv, ea0, ea1):
            return (
                pltpu.async_copy(ea0, dtot0.at[dstv], sem1, add=True),
                pltpu.async_copy(ea1, dtot1.at[dstv], sem1, add=True),
                pltpu.async_copy(ea0, eah.at[pl.ds(h0 * E + base, CH)], sem2),
                pltpu.async_copy(ea1, eah.at[pl.ds((h0 + 1) * E + base, CH)], sem2),
            )

        def pair_body(i, carry2):
            baseA = eo + (2 * i) * CH
            baseB = baseA + CH
            pltpu.sync_copy(src.at[pl.ds(baseA, CH)], srcvA)
            pltpu.sync_copy(dst.at[pl.ds(baseA, CH)], dstvA)
            compute_ea(srcvA, dstvA, ea0A, ea1A)
            dA = fire(baseA, dstvA, ea0A, ea1A)
            pltpu.sync_copy(src.at[pl.ds(baseB, CH)], srcvB)
            pltpu.sync_copy(dst.at[pl.ds(baseB, CH)], dstvB)
            compute_ea(srcvB, dstvB, ea0B, ea1B)
            dB = fire(baseB, dstvB, ea0B, ea1B)
            for d in dA:
                d.wait()
            for d in dB:
                d.wait()
            return carry2

        lax.fori_loop(0, NCHUNK // 2, pair_body, 0)
        plsc.subcore_barrier()
        pltpu.sync_copy(dtot0.at[pl.ds(ro, RPT)],
                        den.at[pl.ds(h0 * NP + ro, RPT)])
        pltpu.sync_copy(dtot1.at[pl.ds(ro, RPT)],
                        den.at[pl.ds((h0 + 1) * NP + ro, RPT)])
        plsc.subcore_barrier()
        return carry

    lax.fori_loop(0, 2, group_body, 0)


def _scB_body(hg, src, dst, eah, zrow,
              out1,
              acc,
              srcvA, dstvA, adjA, rowsA, ea0A, ea1A,
              srcvB, dstvB, adjB, rowsB, ea0B, ea1B,
              semA, semB):
    c = lax.axis_index("c")
    t = lax.axis_index("s")
    eo = t * EPT
    ro = t * RPT
    NH = NCHUNK // 2

    def group_body(j, carry):
        g = 2 * c + j
        h0 = 2 * g
        goff = g * NP
        pltpu.sync_copy(zrow, acc.at[pl.ds(ro, RPT)])
        plsc.subcore_barrier()

        def stage(base, srcv, dstv, adj, ea0, ea1):
            pltpu.sync_copy(src.at[pl.ds(base, CH)], srcv)
            pltpu.sync_copy(dst.at[pl.ds(base, CH)], dstv)
            for s in range(CH // L):
                adj[pl.ds(s * L, L)] = srcv[pl.ds(s * L, L)] + goff
            pltpu.sync_copy(eah.at[pl.ds(h0 * E + base, CH)], ea0)
            pltpu.sync_copy(eah.at[pl.ds((h0 + 1) * E + base, CH)], ea1)

        def scale_scatter(rows, ea0, ea1, dstv):
            def scale_body(eb, carry3):
                for u in range(UNROLL):
                    e = eb * UNROLL + u
                    sp0 = plsc.load_gather(ea0, [jnp.full((L,), 0, jnp.int32) + e])
                    sp1 = plsc.load_gather(ea1, [jnp.full((L,), 0, jnp.int32) + e])
                    for r in range(8):
                        sp = sp0 if r < 4 else sp1
                        sl = pl.ds(r * L, L)
                        rows[e, sl] = rows[e, sl] * sp
                return carry3

            lax.fori_loop(0, CH // UNROLL, scale_body, 0)
            pltpu.sync_copy(rows, acc.at[dstv], add=True)

        stage(eo, srcvA, dstvA, adjA, ea0A, ea1A)
        pltpu.async_copy(hg.at[adjA], rowsA, semA)

        def pair_body(i, carry2):
            baseA = eo + (2 * i) * CH
            baseB = baseA + CH
            stage(baseB, srcvB, dstvB, adjB, ea0B, ea1B)
            pltpu.async_copy(hg.at[adjB], rowsB, semB)
            pltpu.make_async_copy(hg.at[adjA], rowsA, semA).wait()
            scale_scatter(rowsA, ea0A, ea1A, dstvA)

            @pl.when(i < NH - 1)
            def _():
                stage(baseA + 2 * CH, srcvA, dstvA, adjA, ea0A, ea1A)
                pltpu.async_copy(hg.at[adjA], rowsA, semA)

            pltpu.make_async_copy(hg.at[adjB], rowsB, semB).wait()
            scale_scatter(rowsB, ea0B, ea1B, dstvB)
            return carry2

        lax.fori_loop(0, NH, pair_body, 0)
        plsc.subcore_barrier()
        pltpu.sync_copy(acc.at[pl.ds(ro, RPT)],
                        out1.at[pl.ds(goff + ro, RPT)])
        plsc.subcore_barrier()
        return carry

    lax.fori_loop(0, 2, group_body, 0)


@jax.jit
def _sc1(hg, asrcT, adstT, src, dst, bmax, zrow, zflat):
    mesh = plsc.VectorSubcoreMesh(core_axis_name="c", subcore_axis_name="s")
    fA = pl.kernel(
        _scA_body,
        out_type=[
            jax.ShapeDtypeStruct((HEADS * E,), jnp.float32),
            jax.ShapeDtypeStruct((HEADS * NP,), jnp.float32),
        ],
        mesh=mesh,
        compiler_params=pltpu.CompilerParams(needs_layout_passes=False),
        scratch_types=[
            pltpu.VMEM_SHARED((NP,), jnp.float32),
            pltpu.VMEM_SHARED((NP,), jnp.float32),
            pltpu.VMEM((NP,), jnp.float32),
            pltpu.VMEM((NP,), jnp.float32),
            pltpu.VMEM((NP,), jnp.float32),
            pltpu.VMEM((NP,), jnp.float32),
            pltpu.VMEM((CH,), jnp.int32),
            pltpu.VMEM((CH,), jnp.int32),
            pltpu.VMEM((CH,), jnp.float32),
            pltpu.VMEM((CH,), jnp.float32),
            pltpu.VMEM((CH,), jnp.int32),
            pltpu.VMEM((CH,), jnp.int32),
            pltpu.VMEM((CH,), jnp.float32),
            pltpu.VMEM((CH,), jnp.float32),
            pltpu.VMEM((L,), jnp.float32),
            pltpu.SemaphoreType.DMA,
            pltpu.SemaphoreType.DMA,
        ],
    )
    eah, den = fA(asrcT, adstT, src, dst, bmax, zflat)
    fB = pl.kernel(
        _scB_body,
        out_type=[
            jax.ShapeDtypeStruct((G * NP, 128), jnp.float32),
        ],
        mesh=mesh,
        compiler_params=pltpu.CompilerParams(needs_layout_passes=False),
        scratch_types=[
            pltpu.VMEM_SHARED((NP, 128), jnp.float32),
            pltpu.VMEM((CH,), jnp.int32),
            pltpu.VMEM((CH,), jnp.int32),
            pltpu.VMEM((CH,), jnp.int32),
            pltpu.VMEM((CH, 128), jnp.float32),
            pltpu.VMEM((CH,), jnp.float32),
            pltpu.VMEM((CH,), jnp.float32),
            pltpu.VMEM((CH,), jnp.int32),
            pltpu.VMEM((CH,), jnp.int32),
            pltpu.VMEM((CH,), jnp.int32),
            pltpu.VMEM((CH, 128), jnp.float32),
            pltpu.VMEM((CH,), jnp.float32),
            pltpu.VMEM((CH,), jnp.float32),
            pltpu.SemaphoreType.DMA,
            pltpu.SemaphoreType.DMA,
        ],
    )
    (out1f,) = fB(hg, src, dst, eah, zrow)
    return out1f, den


def kernel(x, edge_index, W1, att_src1, att_dst1, b1, W2, att_src2, att_dst2, b2):
    f32 = jnp.float32
    src = edge_index[0]
    dst = edge_index[1]
    xp = jnp.pad(x, ((0, NP - N), (0, 0)))

    As = jnp.zeros((HEADS * HID, HEADS), f32)
    Ad = jnp.zeros((HEADS * HID, HEADS), f32)
    for hh in range(HEADS):
        As = As.at[hh * HID:(hh + 1) * HID, hh].set(att_src1[hh])
        Ad = Ad.at[hh * HID:(hh + 1) * HID, hh].set(att_dst1[hh])

    h, asrc, adst = _tc1(xp, W1, As, Ad)

    gm = asrc.max(0) + adst.max(0)
    bmax = jnp.where(gm > 0, gm, gm * f32(0.2))
    bmax16 = jnp.pad(bmax, (0, 16 - HEADS))
    hg = h.reshape(NP, G, 128).transpose(1, 0, 2).reshape(G * NP, 128)
    asrcT = asrc.T.reshape(-1)
    adstT = adst.T.reshape(-1)
    zrow = jnp.zeros((RPT, 128), f32)
    zflat = jnp.zeros((NP,), f32)

    out1f, denf = _sc1(hg, asrcT, adstT, src, dst, bmax16, zrow, zflat)

    or3 = out1f.reshape(G, NP, 128)
    dent = denf.reshape(G, 2, NP).transpose(0, 2, 1)
    a2m = jnp.stack([att_src2[0], att_dst2[0]], axis=1)
    h2, asd = _tc2(or3, dent, b1.reshape(1, HEADS * HID), W2, a2m)

    as2 = asd[:, 0]
    ad2 = asd[:, 1]
    gm2 = as2.max() + ad2.max()
    bm2 = jnp.where(gm2 > 0, gm2, gm2 * f32(0.2))
    bm2v = jnp.broadcast_to(bm2[None], (L,))
    b2b = jnp.broadcast_to(b2[:, None], (2, L)).reshape(-1)
    h2c = h2.T.reshape(-1)

    (out2,) = _sc2(as2, ad2, h2c, src, dst, bm2v, b2b, zflat)
    return out2.reshape(2, NP).T[:N]


# speedup vs baseline: 34.7959x; 1.7084x over previous
import functools

import jax
import jax.numpy as jnp
from jax import lax
from jax.experimental import pallas as pl
from jax.experimental.pallas import tpu as pltpu
from jax.experimental.pallas import tpu_sc as plsc

N = 10000
NP = 10240
E = 320000
F_IN = 128
HID = 64
HEADS = 8
G = 4
L = 16
TILES = 16
EPT = E // TILES
CH = 80
NCHUNK = EPT // CH
CPS = 10
NSUP = NCHUNK // CPS
NCHG = E // CH
RPT = NP // TILES
UNROLL = 4
BN1 = 512
BN2 = 256


def _tc1_body(x_ref, W1_ref, As_ref, Ad_ref, h_ref, as_ref, ad_ref):
    h = jnp.dot(x_ref[...], W1_ref[...], preferred_element_type=jnp.float32)
    h_ref[...] = h
    as_ref[...] = jnp.dot(h, As_ref[...], preferred_element_type=jnp.float32)
    ad_ref[...] = jnp.dot(h, Ad_ref[...], preferred_element_type=jnp.float32)


def _tc1(xp, W1, As, Ad):
    return pl.pallas_call(
        _tc1_body,
        grid=(NP // BN1,),
        in_specs=[
            pl.BlockSpec((BN1, F_IN), lambda i: (i, 0)),
            pl.BlockSpec((F_IN, HEADS * HID), lambda i: (0, 0)),
            pl.BlockSpec((HEADS * HID, HEADS), lambda i: (0, 0)),
            pl.BlockSpec((HEADS * HID, HEADS), lambda i: (0, 0)),
        ],
        out_specs=[
            pl.BlockSpec((BN1, HEADS * HID), lambda i: (i, 0)),
            pl.BlockSpec((BN1, HEADS), lambda i: (i, 0)),
            pl.BlockSpec((BN1, HEADS), lambda i: (i, 0)),
        ],
        out_shape=[
            jax.ShapeDtypeStruct((NP, HEADS * HID), jnp.float32),
            jax.ShapeDtypeStruct((NP, HEADS), jnp.float32),
            jax.ShapeDtypeStruct((NP, HEADS), jnp.float32),
        ],
    )(xp, W1, As, Ad)


def _tc2_body(or_ref, dent_ref, b1_ref, W2_ref, a2m_ref, h2_ref, asd_ref):
    f32 = jnp.float32
    parts = []
    for g in range(G):
        d0 = jnp.broadcast_to(dent_ref[g, :, 0:1], (BN2, HID))
        d1 = jnp.broadcast_to(dent_ref[g, :, 1:2], (BN2, HID))
        dcat = jnp.concatenate([d0, d1], axis=1)
        parts.append(or_ref[g] / (dcat + f32(1e-16)))
    h1 = jnp.concatenate(parts, axis=1) + b1_ref[...]
    h1 = jnp.where(h1 > 0, h1, jnp.exp(jnp.minimum(h1, 0.0)) - f32(1.0))
    h2 = jnp.dot(h1, W2_ref[...], preferred_element_type=f32)
    h2_ref[...] = h2
    asd_ref[...] = jnp.dot(h2, a2m_ref[...], preferred_element_type=f32)


def _tc2(or3, dent, b1r, W2, a2m):
    return pl.pallas_call(
        _tc2_body,
        grid=(NP // BN2,),
        in_specs=[
            pl.BlockSpec((G, BN2, 2 * HID), lambda i: (0, i, 0)),
            pl.BlockSpec((G, BN2, 2), lambda i: (0, i, 0)),
            pl.BlockSpec((1, HEADS * HID), lambda i: (0, 0)),
            pl.BlockSpec((HEADS * HID, 2), lambda i: (0, 0)),
            pl.BlockSpec((2, 2), lambda i: (0, 0)),
        ],
        out_specs=[
            pl.BlockSpec((BN2, 2), lambda i: (i, 0)),
            pl.BlockSpec((BN2, 2), lambda i: (i, 0)),
        ],
        out_shape=[
            jax.ShapeDtypeStruct((NP, 2), jnp.float32),
            jax.ShapeDtypeStruct((NP, 2), jnp.float32),
        ],
    )(or3, dent, b1r, W2, a2m)


def _sc2_body(as2T, ad2T, h2c, src, dst, bm2, b2b, zflat,
              out2,
              o0s, o1s, d2s,
              as2v, ad2v, h0v, h1v, srcv, dstv, eab, v0b, v1b,
              bm2v, b2v, s0, s1, s2, ob):
    c = lax.axis_index("c")
    t = lax.axis_index("s")
    eo = t * EPT
    ro = t * RPT

    @pl.when(c == 0)
    def _():
        pltpu.sync_copy(as2T, as2v)
        pltpu.sync_copy(ad2T, ad2v)
        pltpu.sync_copy(h2c.at[pl.ds(0, NP)], h0v)
        pltpu.sync_copy(h2c.at[pl.ds(NP, NP)], h1v)
        pltpu.sync_copy(bm2, bm2v)
        pltpu.sync_copy(b2b, b2v)
        pltpu.sync_copy(zflat.at[pl.ds(ro, RPT)], o0s.at[pl.ds(ro, RPT)])
        pltpu.sync_copy(zflat.at[pl.ds(ro, RPT)], o1s.at[pl.ds(ro, RPT)])
        pltpu.sync_copy(zflat.at[pl.ds(ro, RPT)], d2s.at[pl.ds(ro, RPT)])
        plsc.subcore_barrier()
        bmv = bm2v[...]

        def chunk_body(k, carry):
            base = eo + k * CH
            pltpu.sync_copy(src.at[pl.ds(base, CH)], srcv)
            pltpu.sync_copy(dst.at[pl.ds(base, CH)], dstv)
            for s in range(CH // L):
                sv = srcv[pl.ds(s * L, L)]
                dv = dstv[pl.ds(s * L, L)]
                a = plsc.load_gather(as2v, [sv]) + plsc.load_gather(ad2v, [dv])
                a = jnp.where(a > 0, a, a * jnp.float32(0.2))
                ea = jnp.exp(a - bmv)
                eab[pl.ds(s * L, L)] = ea
                v0b[pl.ds(s * L, L)] = ea * plsc.load_gather(h0v, [sv])
                v1b[pl.ds(s * L, L)] = ea * plsc.load_gather(h1v, [sv])
            pltpu.sync_copy(eab, d2s.at[dstv], add=True)
            pltpu.sync_copy(v0b, o0s.at[dstv], add=True)
            pltpu.sync_copy(v1b, o1s.at[dstv], add=True)
            return carry

        lax.fori_loop(0, NCHUNK, chunk_body, 0)
        plsc.subcore_barrier()
        pltpu.sync_copy(o0s.at[pl.ds(ro, RPT)], s0)
        pltpu.sync_copy(o1s.at[pl.ds(ro, RPT)], s1)
        pltpu.sync_copy(d2s.at[pl.ds(ro, RPT)], s2)
        bb0 = b2v[pl.ds(0, L)]
        bb1 = b2v[pl.ds(L, L)]

        def out_body(i, carry):
            sl = pl.ds(i * L, L)
            d = s2[sl] + jnp.float32(1e-16)
            ob[sl] = s0[sl] / d + bb0
            s0[sl] = s1[sl] / d + bb1
            return carry

        lax.fori_loop(0, RPT // L, out_body, 0)
        pltpu.sync_copy(ob, out2.at[pl.ds(ro, RPT)])
        pltpu.sync_copy(s0, out2.at[pl.ds(NP + ro, RPT)])


def _sc2(as2T, ad2T, h2c, src, dst, bm2, b2b, zflat):
    mesh = plsc.VectorSubcoreMesh(core_axis_name="c", subcore_axis_name="s")
    f = pl.kernel(
        _sc2_body,
        out_type=[jax.ShapeDtypeStruct((2 * NP,), jnp.float32)],
        mesh=mesh,
        compiler_params=pltpu.CompilerParams(needs_layout_passes=False),
        scratch_types=[
            pltpu.VMEM_SHARED((NP,), jnp.float32),
            pltpu.VMEM_SHARED((NP,), jnp.float32),
            pltpu.VMEM_SHARED((NP,), jnp.float32),
            pltpu.VMEM((NP,), jnp.float32),
            pltpu.VMEM((NP,), jnp.float32),
            pltpu.VMEM((NP,), jnp.float32),
            pltpu.VMEM((NP,), jnp.float32),
            pltpu.VMEM((CH,), jnp.int32),
            pltpu.VMEM((CH,), jnp.int32),
            pltpu.VMEM((CH,), jnp.float32),
            pltpu.VMEM((CH,), jnp.float32),
            pltpu.VMEM((CH,), jnp.float32),
            pltpu.VMEM((L,), jnp.float32),
            pltpu.VMEM((2 * L,), jnp.float32),
            pltpu.VMEM((RPT,), jnp.float32),
            pltpu.VMEM((RPT,), jnp.float32),
            pltpu.VMEM((RPT,), jnp.float32),
            pltpu.VMEM((RPT,), jnp.float32),
        ],
    )
    return f(as2T, ad2T, h2c, src, dst, bm2, b2b, zflat)


def _scA_body(asrcT, adstT, src, dst, bmax, zflat,
              eah, den,
              dtot0, dtot1,
              asrc0v, asrc1v, adst0v, adst1v,
              srcS, dstS, dstCA, dstCB, ea0A, ea1A, ea0B, ea1B,
              bmaxv, sem1, sem2):
    c = lax.axis_index("c")
    t = lax.axis_index("s")
    eo = t * EPT
    ro = t * RPT
    pltpu.sync_copy(bmax, bmaxv)

    def group_body(j, carry):
        g = 2 * c + j
        h0 = 2 * g
        pltpu.sync_copy(asrcT.at[pl.ds(h0 * NP, NP)], asrc0v)
        pltpu.sync_copy(asrcT.at[pl.ds((h0 + 1) * NP, NP)], asrc1v)
        pltpu.sync_copy(adstT.at[pl.ds(h0 * NP, NP)], adst0v)
        pltpu.sync_copy(adstT.at[pl.ds((h0 + 1) * NP, NP)], adst1v)
        pltpu.sync_copy(zflat.at[pl.ds(ro, RPT)], dtot0.at[pl.ds(ro, RPT)])
        pltpu.sync_copy(zflat.at[pl.ds(ro, RPT)], dtot1.at[pl.ds(ro, RPT)])
        plsc.subcore_barrier()
        b0 = plsc.load_gather(bmaxv, [jnp.full((L,), 0, jnp.int32) + h0])
        b1 = plsc.load_gather(bmaxv, [jnp.full((L,), 1, jnp.int32) + h0])

        def compute_ea(cc, dstC, ea0, ea1):
            for s in range(CH // L):
                sv = srcS[pl.ds(cc * CH + s * L, L)]
                dv = dstS[pl.ds(cc * CH + s * L, L)]
                dstC[pl.ds(s * L, L)] = dv
                a0 = plsc.load_gather(asrc0v, [sv]) + plsc.load_gather(adst0v, [dv])
                a0 = jnp.where(a0 > 0, a0, a0 * jnp.float32(0.2))
                ea0[pl.ds(s * L, L)] = jnp.exp(a0 - b0)
                a1 = plsc.load_gather(asrc1v, [sv]) + plsc.load_gather(adst1v, [dv])
                a1 = jnp.where(a1 > 0, a1, a1 * jnp.float32(0.2))
                ea1[pl.ds(s * L, L)] = jnp.exp(a1 - b1)

        def super_body(sidx, carry2):
            ebase = eo + sidx * (CPS * CH)
            pltpu.sync_copy(src.at[pl.ds(ebase, CPS * CH)], srcS)
            pltpu.sync_copy(dst.at[pl.ds(ebase, CPS * CH)], dstS)
            prev = ()
            for cc in range(CPS):
                dstC, ea0, ea1 = ((dstCA, ea0A, ea1A) if cc % 2 == 0
                                  else (dstCB, ea0B, ea1B))
                compute_ea(cc, dstC, ea0, ea1)
                for d in prev:
                    d.wait()
                prev = (
                    pltpu.async_copy(ea0, dtot0.at[dstC], sem1, add=True),
                    pltpu.async_copy(ea1, dtot1.at[dstC], sem1, add=True),
                    pltpu.async_copy(
                        ea0, eah.at[pl.ds(h0 * E + ebase + cc * CH, CH)], sem2),
                    pltpu.async_copy(
                        ea1, eah.at[pl.ds((h0 + 1) * E + ebase + cc * CH, CH)], sem2),
                )
            for d in prev:
                d.wait()
            return carry2

        lax.fori_loop(0, NSUP, super_body, 0)
        plsc.subcore_barrier()
        pltpu.sync_copy(dtot0.at[pl.ds(ro, RPT)],
                        den.at[pl.ds(h0 * NP + ro, RPT)])
        pltpu.sync_copy(dtot1.at[pl.ds(ro, RPT)],
                        den.at[pl.ds((h0 + 1) * NP + ro, RPT)])
        plsc.subcore_barrier()
        return carry

    lax.fori_loop(0, 2, group_body, 0)


def _scB_body(hg, src, dst, eah, zrow,
              out1,
              acc,
              srcS, dstS, ea0S, ea1S,
              adjA, rowsA, dstCA, adjB, rowsB, dstCB,
              semA, semB):
    c = lax.axis_index("c")
    t = lax.axis_index("s")
    eo = t * EPT
    ro = t * RPT

    def group_body(j, carry):
        g = 2 * c + j
        h0 = 2 * g
        goff = g * NP
        pltpu.sync_copy(zrow, acc.at[pl.ds(ro, RPT)])
        plsc.subcore_barrier()

        def build_adj(cc, adj, dstC):
            for s in range(CH // L):
                adj[pl.ds(s * L, L)] = srcS[pl.ds(cc * CH + s * L, L)] + goff
                dstC[pl.ds(s * L, L)] = dstS[pl.ds(cc * CH + s * L, L)]

        def scale_scatter(cc, rows, dstC):
            def scale_body(eb, carry3):
                for u in range(UNROLL):
                    e = eb * UNROLL + u
                    ev = jnp.full((L,), cc * CH, jnp.int32) + e
                    sp0 = plsc.load_gather(ea0S, [ev])
                    sp1 = plsc.load_gather(ea1S, [ev])
                    for r in range(8):
                        sp = sp0 if r < 4 else sp1
                        sl = pl.ds(r * L, L)
                        rows[e, sl] = rows[e, sl] * sp
                return carry3

            lax.fori_loop(0, CH // UNROLL, scale_body, 0)
            pltpu.sync_copy(rows, acc.at[dstC], add=True)

        def super_body(sidx, carry2):
            ebase = eo + sidx * (CPS * CH)
            pltpu.sync_copy(src.at[pl.ds(ebase, CPS * CH)], srcS)
            pltpu.sync_copy(dst.at[pl.ds(ebase, CPS * CH)], dstS)
            pltpu.sync_copy(eah.at[pl.ds(h0 * E + ebase, CPS * CH)], ea0S)
            pltpu.sync_copy(eah.at[pl.ds((h0 + 1) * E + ebase, CPS * CH)], ea1S)
            build_adj(0, adjA, dstCA)
            pltpu.async_copy(hg.at[adjA], rowsA, semA)
            for m in range(CPS // 2):
                cA = 2 * m
                cB = cA + 1
                build_adj(cB, adjB, dstCB)
                pltpu.async_copy(hg.at[adjB], rowsB, semB)
                pltpu.make_async_copy(hg.at[adjA], rowsA, semA).wait()
                scale_scatter(cA, rowsA, dstCA)
                if m < CPS // 2 - 1:
                    build_adj(cA + 2, adjA, dstCA)
                    pltpu.async_copy(hg.at[adjA], rowsA, semA)
                pltpu.make_async_copy(hg.at[adjB], rowsB, semB).wait()
                scale_scatter(cB, rowsB, dstCB)
            return carry2

        lax.fori_loop(0, NSUP, super_body, 0)
        plsc.subcore_barrier()
        pltpu.sync_copy(acc.at[pl.ds(ro, RPT)],
                        out1.at[pl.ds(goff + ro, RPT)])
        plsc.subcore_barrier()
        return carry

    lax.fori_loop(0, 2, group_body, 0)


@jax.jit
def _sc1(hg, asrcT, adstT, src, dst, bmax, zrow, zflat):
    mesh = plsc.VectorSubcoreMesh(core_axis_name="c", subcore_axis_name="s")
    fA = pl.kernel(
        _scA_body,
        out_type=[
            jax.ShapeDtypeStruct((HEADS * E,), jnp.float32),
            jax.ShapeDtypeStruct((HEADS * NP,), jnp.float32),
        ],
        mesh=mesh,
        compiler_params=pltpu.CompilerParams(needs_layout_passes=False),
        scratch_types=[
            pltpu.VMEM_SHARED((NP,), jnp.float32),
            pltpu.VMEM_SHARED((NP,), jnp.float32),
            pltpu.VMEM((NP,), jnp.float32),
            pltpu.VMEM((NP,), jnp.float32),
            pltpu.VMEM((NP,), jnp.float32),
            pltpu.VMEM((NP,), jnp.float32),
            pltpu.VMEM((CPS * CH,), jnp.int32),
            pltpu.VMEM((CPS * CH,), jnp.int32),
            pltpu.VMEM((CH,), jnp.int32),
            pltpu.VMEM((CH,), jnp.int32),
            pltpu.VMEM((CH,), jnp.float32),
            pltpu.VMEM((CH,), jnp.float32),
            pltpu.VMEM((CH,), jnp.float32),
            pltpu.VMEM((CH,), jnp.float32),
            pltpu.VMEM((L,), jnp.float32),
            pltpu.SemaphoreType.DMA,
            pltpu.SemaphoreType.DMA,
        ],
    )
    eah, den = fA(asrcT, adstT, src, dst, bmax, zflat)
    fB = pl.kernel(
        _scB_body,
        out_type=[
            jax.ShapeDtypeStruct((G * NP, 128), jnp.float32),
        ],
        mesh=mesh,
        compiler_params=pltpu.CompilerParams(needs_layout_passes=False),
        scratch_types=[
            pltpu.VMEM_SHARED((NP, 128), jnp.float32),
            pltpu.VMEM((CPS * CH,), jnp.int32),
            pltpu.VMEM((CPS * CH,), jnp.int32),
            pltpu.VMEM((CPS * CH,), jnp.float32),
            pltpu.VMEM((CPS * CH,), jnp.float32),
            pltpu.VMEM((CH,), jnp.int32),
            pltpu.VMEM((CH, 128), jnp.float32),
            pltpu.VMEM((CH,), jnp.int32),
            pltpu.VMEM((CH,), jnp.int32),
            pltpu.VMEM((CH, 128), jnp.float32),
            pltpu.VMEM((CH,), jnp.int32),
            pltpu.SemaphoreType.DMA,
            pltpu.SemaphoreType.DMA,
        ],
    )
    (out1f,) = fB(hg, src, dst, eah, zrow)
    return out1f, den


def kernel(x, edge_index, W1, att_src1, att_dst1, b1, W2, att_src2, att_dst2, b2):
    f32 = jnp.float32
    src = edge_index[0]
    dst = edge_index[1]
    xp = jnp.pad(x, ((0, NP - N), (0, 0)))

    As = jnp.zeros((HEADS * HID, HEADS), f32)
    Ad = jnp.zeros((HEADS * HID, HEADS), f32)
    for hh in range(HEADS):
        As = As.at[hh * HID:(hh + 1) * HID, hh].set(att_src1[hh])
        Ad = Ad.at[hh * HID:(hh + 1) * HID, hh].set(att_dst1[hh])

    h, asrc, adst = _tc1(xp, W1, As, Ad)

    gm = asrc.max(0) + adst.max(0)
    bmax = jnp.where(gm > 0, gm, gm * f32(0.2))
    bmax16 = jnp.pad(bmax, (0, 16 - HEADS))
    hg = h.reshape(NP, G, 128).transpose(1, 0, 2).reshape(G * NP, 128)
    asrcT = asrc.T.reshape(-1)
    adstT = adst.T.reshape(-1)
    zrow = jnp.zeros((RPT, 128), f32)
    zflat = jnp.zeros((NP,), f32)

    out1f, denf = _sc1(hg, asrcT, adstT, src, dst, bmax16, zrow, zflat)

    or3 = out1f.reshape(G, NP, 128)
    dent = denf.reshape(G, 2, NP).transpose(0, 2, 1)
    a2m = jnp.stack([att_src2[0], att_dst2[0]], axis=1)
    h2, asd = _tc2(or3, dent, b1.reshape(1, HEADS * HID), W2, a2m)

    as2 = asd[:, 0]
    ad2 = asd[:, 1]
    gm2 = as2.max() + ad2.max()
    bm2 = jnp.where(gm2 > 0, gm2, gm2 * f32(0.2))
    bm2v = jnp.broadcast_to(bm2[None], (L,))
    b2b = jnp.broadcast_to(b2[:, None], (2, L)).reshape(-1)
    h2c = h2.T.reshape(-1)

    (out2,) = _sc2(as2, ad2, h2c, src, dst, bm2v, b2b, zflat)
    return out2.reshape(2, NP).T[:N]


# speedup vs baseline: 41.6582x; 1.1972x over previous
import functools

import jax
import jax.numpy as jnp
from jax import lax
from jax.experimental import pallas as pl
from jax.experimental.pallas import tpu as pltpu
from jax.experimental.pallas import tpu_sc as plsc

N = 10000
NP = 10240
E = 320000
F_IN = 128
HID = 64
HEADS = 8
G = 4
L = 16
TILES = 16
EPT = E // TILES
CH = 80
NCHUNK = EPT // CH
CPS = 10
NSUP = NCHUNK // CPS
NCHG = E // CH
RPT = NP // TILES
UNROLL = 4
BN1 = 512
BN2 = 256


def _tc1_body(x_ref, W1_ref, As_ref, Ad_ref, h_ref, as_ref, ad_ref):
    h = jnp.dot(x_ref[...], W1_ref[...], preferred_element_type=jnp.float32)
    h_ref[...] = h
    as_ref[...] = jnp.dot(h, As_ref[...], preferred_element_type=jnp.float32)
    ad_ref[...] = jnp.dot(h, Ad_ref[...], preferred_element_type=jnp.float32)


def _tc1(xp, W1, As, Ad):
    return pl.pallas_call(
        _tc1_body,
        grid=(NP // BN1,),
        in_specs=[
            pl.BlockSpec((BN1, F_IN), lambda i: (i, 0)),
            pl.BlockSpec((F_IN, HEADS * HID), lambda i: (0, 0)),
            pl.BlockSpec((HEADS * HID, HEADS), lambda i: (0, 0)),
            pl.BlockSpec((HEADS * HID, HEADS), lambda i: (0, 0)),
        ],
        out_specs=[
            pl.BlockSpec((BN1, HEADS * HID), lambda i: (i, 0)),
            pl.BlockSpec((BN1, HEADS), lambda i: (i, 0)),
            pl.BlockSpec((BN1, HEADS), lambda i: (i, 0)),
        ],
        out_shape=[
            jax.ShapeDtypeStruct((NP, HEADS * HID), jnp.float32),
            jax.ShapeDtypeStruct((NP, HEADS), jnp.float32),
            jax.ShapeDtypeStruct((NP, HEADS), jnp.float32),
        ],
    )(xp, W1, As, Ad)


def _tc2_body(or_ref, dent_ref, b1_ref, W2_ref, a2m_ref, h2_ref, asd_ref):
    f32 = jnp.float32
    parts = []
    for g in range(G):
        d0 = jnp.broadcast_to(dent_ref[g, :, 0:1], (BN2, HID))
        d1 = jnp.broadcast_to(dent_ref[g, :, 1:2], (BN2, HID))
        dcat = jnp.concatenate([d0, d1], axis=1)
        parts.append(or_ref[g] / (dcat + f32(1e-16)))
    h1 = jnp.concatenate(parts, axis=1) + b1_ref[...]
    h1 = jnp.where(h1 > 0, h1, jnp.exp(jnp.minimum(h1, 0.0)) - f32(1.0))
    h2 = jnp.dot(h1, W2_ref[...], preferred_element_type=f32)
    h2_ref[...] = h2
    asd_ref[...] = jnp.dot(h2, a2m_ref[...], preferred_element_type=f32)


def _tc2(or3, dent, b1r, W2, a2m):
    return pl.pallas_call(
        _tc2_body,
        grid=(NP // BN2,),
        in_specs=[
            pl.BlockSpec((G, BN2, 2 * HID), lambda i: (0, i, 0)),
            pl.BlockSpec((G, BN2, 2), lambda i: (0, i, 0)),
            pl.BlockSpec((1, HEADS * HID), lambda i: (0, 0)),
            pl.BlockSpec((HEADS * HID, 2), lambda i: (0, 0)),
            pl.BlockSpec((2, 2), lambda i: (0, 0)),
        ],
        out_specs=[
            pl.BlockSpec((BN2, 2), lambda i: (i, 0)),
            pl.BlockSpec((BN2, 2), lambda i: (i, 0)),
        ],
        out_shape=[
            jax.ShapeDtypeStruct((NP, 2), jnp.float32),
            jax.ShapeDtypeStruct((NP, 2), jnp.float32),
        ],
    )(or3, dent, b1r, W2, a2m)


def _sc2_body(as2T, ad2T, h2c, src, dst, bm2, b2b, zflat,
              out2,
              o0s, o1s, d2s,
              as2v, ad2v, h0v, h1v, srcS, dstS,
              dstCA, eabA, v0bA, v1bA, dstCB, eabB, v0bB, v1bB,
              bm2v, b2v, s0, s1, s2, ob, sem1):
    c = lax.axis_index("c")
    t = lax.axis_index("s")
    eo = t * EPT
    ro = t * RPT

    @pl.when(c == 0)
    def _():
        pltpu.sync_copy(as2T, as2v)
        pltpu.sync_copy(ad2T, ad2v)
        pltpu.sync_copy(h2c.at[pl.ds(0, NP)], h0v)
        pltpu.sync_copy(h2c.at[pl.ds(NP, NP)], h1v)
        pltpu.sync_copy(bm2, bm2v)
        pltpu.sync_copy(b2b, b2v)
        pltpu.sync_copy(zflat.at[pl.ds(ro, RPT)], o0s.at[pl.ds(ro, RPT)])
        pltpu.sync_copy(zflat.at[pl.ds(ro, RPT)], o1s.at[pl.ds(ro, RPT)])
        pltpu.sync_copy(zflat.at[pl.ds(ro, RPT)], d2s.at[pl.ds(ro, RPT)])
        plsc.subcore_barrier()
        bmv = bm2v[...]

        def super_body(sidx, carry):
            ebase = eo + sidx * (CPS * CH)
            pltpu.sync_copy(src.at[pl.ds(ebase, CPS * CH)], srcS)
            pltpu.sync_copy(dst.at[pl.ds(ebase, CPS * CH)], dstS)
            prev = ()
            for cc in range(CPS):
                dstC, eab, v0b, v1b = ((dstCA, eabA, v0bA, v1bA) if cc % 2 == 0
                                       else (dstCB, eabB, v0bB, v1bB))
                for s in range(CH // L):
                    sv = srcS[pl.ds(cc * CH + s * L, L)]
                    dv = dstS[pl.ds(cc * CH + s * L, L)]
                    dstC[pl.ds(s * L, L)] = dv
                    a = plsc.load_gather(as2v, [sv]) + plsc.load_gather(ad2v, [dv])
                    a = jnp.where(a > 0, a, a * jnp.float32(0.2))
                    ea = jnp.exp(a - bmv)
                    eab[pl.ds(s * L, L)] = ea
                    v0b[pl.ds(s * L, L)] = ea * plsc.load_gather(h0v, [sv])
                    v1b[pl.ds(s * L, L)] = ea * plsc.load_gather(h1v, [sv])
                for d in prev:
                    d.wait()
                prev = (
                    pltpu.async_copy(eab, d2s.at[dstC], sem1, add=True),
                    pltpu.async_copy(v0b, o0s.at[dstC], sem1, add=True),
                    pltpu.async_copy(v1b, o1s.at[dstC], sem1, add=True),
                )
            for d in prev:
                d.wait()
            return carry

        lax.fori_loop(0, NSUP, super_body, 0)
        plsc.subcore_barrier()
        pltpu.sync_copy(o0s.at[pl.ds(ro, RPT)], s0)
        pltpu.sync_copy(o1s.at[pl.ds(ro, RPT)], s1)
        pltpu.sync_copy(d2s.at[pl.ds(ro, RPT)], s2)
        bb0 = b2v[pl.ds(0, L)]
        bb1 = b2v[pl.ds(L, L)]

        def out_body(i, carry):
            sl = pl.ds(i * L, L)
            d = s2[sl] + jnp.float32(1e-16)
            ob[sl] = s0[sl] / d + bb0
            s0[sl] = s1[sl] / d + bb1
            return carry

        lax.fori_loop(0, RPT // L, out_body, 0)
        pltpu.sync_copy(ob, out2.at[pl.ds(ro, RPT)])
        pltpu.sync_copy(s0, out2.at[pl.ds(NP + ro, RPT)])


def _sc2(as2T, ad2T, h2c, src, dst, bm2, b2b, zflat):
    mesh = plsc.VectorSubcoreMesh(core_axis_name="c", subcore_axis_name="s")
    f = pl.kernel(
        _sc2_body,
        out_type=[jax.ShapeDtypeStruct((2 * NP,), jnp.float32)],
        mesh=mesh,
        compiler_params=pltpu.CompilerParams(needs_layout_passes=False),
        scratch_types=[
            pltpu.VMEM_SHARED((NP,), jnp.float32),
            pltpu.VMEM_SHARED((NP,), jnp.float32),
            pltpu.VMEM_SHARED((NP,), jnp.float32),
            pltpu.VMEM((NP,), jnp.float32),
            pltpu.VMEM((NP,), jnp.float32),
            pltpu.VMEM((NP,), jnp.float32),
            pltpu.VMEM((NP,), jnp.float32),
            pltpu.VMEM((CPS * CH,), jnp.int32),
            pltpu.VMEM((CPS * CH,), jnp.int32),
            pltpu.VMEM((CH,), jnp.int32),
            pltpu.VMEM((CH,), jnp.float32),
            pltpu.VMEM((CH,), jnp.float32),
            pltpu.VMEM((CH,), jnp.float32),
            pltpu.VMEM((CH,), jnp.int32),
            pltpu.VMEM((CH,), jnp.float32),
            pltpu.VMEM((CH,), jnp.float32),
            pltpu.VMEM((CH,), jnp.float32),
            pltpu.VMEM((L,), jnp.float32),
            pltpu.VMEM((2 * L,), jnp.float32),
            pltpu.VMEM((RPT,), jnp.float32),
            pltpu.VMEM((RPT,), jnp.float32),
            pltpu.VMEM((RPT,), jnp.float32),
            pltpu.VMEM((RPT,), jnp.float32),
            pltpu.SemaphoreType.DMA,
        ],
    )
    return f(as2T, ad2T, h2c, src, dst, bm2, b2b, zflat)


def _scA_body(asrcT, adstT, src, dst, bmax, zflat,
              eah, den,
              dtot0, dtot1,
              asrc0v, asrc1v, adst0v, adst1v,
              srcS, dstS, dstCA, dstCB, ea0A, ea1A, ea0B, ea1B,
              bmaxv, sem1, sem2):
    c = lax.axis_index("c")
    t = lax.axis_index("s")
    eo = t * EPT
    ro = t * RPT
    pltpu.sync_copy(bmax, bmaxv)

    def group_body(j, carry):
        g = 2 * c + j
        h0 = 2 * g
        pltpu.sync_copy(asrcT.at[pl.ds(h0 * NP, NP)], asrc0v)
        pltpu.sync_copy(asrcT.at[pl.ds((h0 + 1) * NP, NP)], asrc1v)
        pltpu.sync_copy(adstT.at[pl.ds(h0 * NP, NP)], adst0v)
        pltpu.sync_copy(adstT.at[pl.ds((h0 + 1) * NP, NP)], adst1v)
        pltpu.sync_copy(zflat.at[pl.ds(ro, RPT)], dtot0.at[pl.ds(ro, RPT)])
        pltpu.sync_copy(zflat.at[pl.ds(ro, RPT)], dtot1.at[pl.ds(ro, RPT)])
        plsc.subcore_barrier()
        b0 = plsc.load_gather(bmaxv, [jnp.full((L,), 0, jnp.int32) + h0])
        b1 = plsc.load_gather(bmaxv, [jnp.full((L,), 1, jnp.int32) + h0])

        def compute_ea(cc, dstC, ea0, ea1):
            for s in range(CH // L):
                sv = srcS[pl.ds(cc * CH + s * L, L)]
                dv = dstS[pl.ds(cc * CH + s * L, L)]
                dstC[pl.ds(s * L, L)] = dv
                a0 = plsc.load_gather(asrc0v, [sv]) + plsc.load_gather(adst0v, [dv])
                a0 = jnp.where(a0 > 0, a0, a0 * jnp.float32(0.2))
                ea0[pl.ds(s * L, L)] = jnp.exp(a0 - b0)
                a1 = plsc.load_gather(asrc1v, [sv]) + plsc.load_gather(adst1v, [dv])
                a1 = jnp.where(a1 > 0, a1, a1 * jnp.float32(0.2))
                ea1[pl.ds(s * L, L)] = jnp.exp(a1 - b1)

        def super_body(sidx, carry2):
            ebase = eo + sidx * (CPS * CH)
            pltpu.sync_copy(src.at[pl.ds(ebase, CPS * CH)], srcS)
            pltpu.sync_copy(dst.at[pl.ds(ebase, CPS * CH)], dstS)
            prev = ()
            for cc in range(CPS):
                dstC, ea0, ea1 = ((dstCA, ea0A, ea1A) if cc % 2 == 0
                                  else (dstCB, ea0B, ea1B))
                compute_ea(cc, dstC, ea0, ea1)
                for d in prev:
                    d.wait()
                prev = (
                    pltpu.async_copy(ea0, dtot0.at[dstC], sem1, add=True),
                    pltpu.async_copy(ea1, dtot1.at[dstC], sem1, add=True),
                    pltpu.async_copy(
                        ea0, eah.at[pl.ds(h0 * E + ebase + cc * CH, CH)], sem2),
                    pltpu.async_copy(
                        ea1, eah.at[pl.ds((h0 + 1) * E + ebase + cc * CH, CH)], sem2),
                )
            for d in prev:
                d.wait()
            return carry2

        lax.fori_loop(0, NSUP, super_body, 0)
        plsc.subcore_barrier()
        pltpu.sync_copy(dtot0.at[pl.ds(ro, RPT)],
                        den.at[pl.ds(h0 * NP + ro, RPT)])
        pltpu.sync_copy(dtot1.at[pl.ds(ro, RPT)],
                        den.at[pl.ds((h0 + 1) * NP + ro, RPT)])
        plsc.subcore_barrier()
        return carry

    lax.fori_loop(0, 2, group_body, 0)


def _scB_body(hg, src, dst, eah, zrow,
              out1,
              acc,
              srcS, dstS, ea0S, ea1S,
              adjA, rowsA, dstCA, adjB, rowsB, dstCB,
              semA, semB):
    c = lax.axis_index("c")
    t = lax.axis_index("s")
    eo = t * EPT
    ro = t * RPT

    def group_body(j, carry):
        g = 2 * c + j
        h0 = 2 * g
        goff = g * NP
        pltpu.sync_copy(zrow, acc.at[pl.ds(ro, RPT)])
        plsc.subcore_barrier()

        def build_adj(cc, adj, dstC):
            for s in range(CH // L):
                adj[pl.ds(s * L, L)] = srcS[pl.ds(cc * CH + s * L, L)] + goff
                dstC[pl.ds(s * L, L)] = dstS[pl.ds(cc * CH + s * L, L)]

        def scale_scatter(cc, rows, dstC):
            def scale_body(eb, carry3):
                for u in range(UNROLL):
                    e = eb * UNROLL + u
                    ev = jnp.full((L,), cc * CH, jnp.int32) + e
                    sp0 = plsc.load_gather(ea0S, [ev])
                    sp1 = plsc.load_gather(ea1S, [ev])
                    for r in range(8):
                        sp = sp0 if r < 4 else sp1
                        sl = pl.ds(r * L, L)
                        rows[e, sl] = rows[e, sl] * sp
                return carry3

            lax.fori_loop(0, CH // UNROLL, scale_body, 0)
            pltpu.sync_copy(rows, acc.at[dstC], add=True)

        def super_body(sidx, carry2):
            ebase = eo + sidx * (CPS * CH)
            pltpu.sync_copy(src.at[pl.ds(ebase, CPS * CH)], srcS)
            pltpu.sync_copy(dst.at[pl.ds(ebase, CPS * CH)], dstS)
            pltpu.sync_copy(eah.at[pl.ds(h0 * E + ebase, CPS * CH)], ea0S)
            pltpu.sync_copy(eah.at[pl.ds((h0 + 1) * E + ebase, CPS * CH)], ea1S)
            build_adj(0, adjA, dstCA)
            pltpu.async_copy(hg.at[adjA], rowsA, semA)
            for m in range(CPS // 2):
                cA = 2 * m
                cB = cA + 1
                build_adj(cB, adjB, dstCB)
                pltpu.async_copy(hg.at[adjB], rowsB, semB)
                pltpu.make_async_copy(hg.at[adjA], rowsA, semA).wait()
                scale_scatter(cA, rowsA, dstCA)
                if m < CPS // 2 - 1:
                    build_adj(cA + 2, adjA, dstCA)
                    pltpu.async_copy(hg.at[adjA], rowsA, semA)
                pltpu.make_async_copy(hg.at[adjB], rowsB, semB).wait()
                scale_scatter(cB, rowsB, dstCB)
            return carry2

        lax.fori_loop(0, NSUP, super_body, 0)
        plsc.subcore_barrier()
        pltpu.sync_copy(acc.at[pl.ds(ro, RPT)],
                        out1.at[pl.ds(goff + ro, RPT)])
        plsc.subcore_barrier()
        return carry

    lax.fori_loop(0, 2, group_body, 0)


@jax.jit
def _sc1(hg, asrcT, adstT, src, dst, bmax, zrow, zflat):
    mesh = plsc.VectorSubcoreMesh(core_axis_name="c", subcore_axis_name="s")
    fA = pl.kernel(
        _scA_body,
        out_type=[
            jax.ShapeDtypeStruct((HEADS * E,), jnp.float32),
            jax.ShapeDtypeStruct((HEADS * NP,), jnp.float32),
        ],
        mesh=mesh,
        compiler_params=pltpu.CompilerParams(needs_layout_passes=False),
        scratch_types=[
            pltpu.VMEM_SHARED((NP,), jnp.float32),
            pltpu.VMEM_SHARED((NP,), jnp.float32),
            pltpu.VMEM((NP,), jnp.float32),
            pltpu.VMEM((NP,), jnp.float32),
            pltpu.VMEM((NP,), jnp.float32),
            pltpu.VMEM((NP,), jnp.float32),
            pltpu.VMEM((CPS * CH,), jnp.int32),
            pltpu.VMEM((CPS * CH,), jnp.int32),
            pltpu.VMEM((CH,), jnp.int32),
            pltpu.VMEM((CH,), jnp.int32),
            pltpu.VMEM((CH,), jnp.float32),
            pltpu.VMEM((CH,), jnp.float32),
            pltpu.VMEM((CH,), jnp.float32),
            pltpu.VMEM((CH,), jnp.float32),
            pltpu.VMEM((L,), jnp.float32),
            pltpu.SemaphoreType.DMA,
            pltpu.SemaphoreType.DMA,
        ],
    )
    eah, den = fA(asrcT, adstT, src, dst, bmax, zflat)
    fB = pl.kernel(
        _scB_body,
        out_type=[
            jax.ShapeDtypeStruct((G * NP, 128), jnp.float32),
        ],
        mesh=mesh,
        compiler_params=pltpu.CompilerParams(needs_layout_passes=False),
        scratch_types=[
            pltpu.VMEM_SHARED((NP, 128), jnp.float32),
            pltpu.VMEM((CPS * CH,), jnp.int32),
            pltpu.VMEM((CPS * CH,), jnp.int32),
            pltpu.VMEM((CPS * CH,), jnp.float32),
            pltpu.VMEM((CPS * CH,), jnp.float32),
            pltpu.VMEM((CH,), jnp.int32),
            pltpu.VMEM((CH, 128), jnp.float32),
            pltpu.VMEM((CH,), jnp.int32),
            pltpu.VMEM((CH,), jnp.int32),
            pltpu.VMEM((CH, 128), jnp.float32),
            pltpu.VMEM((CH,), jnp.int32),
            pltpu.SemaphoreType.DMA,
            pltpu.SemaphoreType.DMA,
        ],
    )
    (out1f,) = fB(hg, src, dst, eah, zrow)
    return out1f, den


def kernel(x, edge_index, W1, att_src1, att_dst1, b1, W2, att_src2, att_dst2, b2):
    f32 = jnp.float32
    src = edge_index[0]
    dst = edge_index[1]
    xp = jnp.pad(x, ((0, NP - N), (0, 0)))

    As = jnp.zeros((HEADS * HID, HEADS), f32)
    Ad = jnp.zeros((HEADS * HID, HEADS), f32)
    for hh in range(HEADS):
        As = As.at[hh * HID:(hh + 1) * HID, hh].set(att_src1[hh])
        Ad = Ad.at[hh * HID:(hh + 1) * HID, hh].set(att_dst1[hh])

    h, asrc, adst = _tc1(xp, W1, As, Ad)

    gm = asrc.max(0) + adst.max(0)
    bmax = jnp.where(gm > 0, gm, gm * f32(0.2))
    bmax16 = jnp.pad(bmax, (0, 16 - HEADS))
    hg = h.reshape(NP, G, 128).transpose(1, 0, 2).reshape(G * NP, 128)
    asrcT = asrc.T.reshape(-1)
    adstT = adst.T.reshape(-1)
    zrow = jnp.zeros((RPT, 128), f32)
    zflat = jnp.zeros((NP,), f32)

    out1f, denf = _sc1(hg, asrcT, adstT, src, dst, bmax16, zrow, zflat)

    or3 = out1f.reshape(G, NP, 128)
    dent = denf.reshape(G, 2, NP).transpose(0, 2, 1)
    a2m = jnp.stack([att_src2[0], att_dst2[0]], axis=1)
    h2, asd = _tc2(or3, dent, b1.reshape(1, HEADS * HID), W2, a2m)

    as2 = asd[:, 0]
    ad2 = asd[:, 1]
    gm2 = as2.max() + ad2.max()
    bm2 = jnp.where(gm2 > 0, gm2, gm2 * f32(0.2))
    bm2v = jnp.broadcast_to(bm2[None], (L,))
    b2b = jnp.broadcast_to(b2[:, None], (2, L)).reshape(-1)
    h2c = h2.T.reshape(-1)

    (out2,) = _sc2(as2, ad2, h2c, src, dst, bm2v, b2b, zflat)
    return out2.reshape(2, NP).T[:N]


# speedup vs baseline: 42.1395x; 1.0116x over previous
import functools

import jax
import jax.numpy as jnp
from jax import lax
from jax.experimental import pallas as pl
from jax.experimental.pallas import tpu as pltpu
from jax.experimental.pallas import tpu_sc as plsc

N = 10000
NP = 10240
E = 320000
F_IN = 128
HID = 64
HEADS = 8
G = 4
L = 16
TILES = 16
EPT = E // TILES
CH = 80
NCHUNK = EPT // CH
CPS = 10
NSUP = NCHUNK // CPS
NCHG = E // CH
RPT = NP // TILES
UNROLL = 4
BN1 = 512
BN2 = 256


def _tc1_body(x_ref, W1_ref, As_ref, Ad_ref, h_ref, as_ref, ad_ref):
    h = jnp.dot(x_ref[...], W1_ref[...], preferred_element_type=jnp.float32)
    h_ref[...] = h
    as_ref[...] = jnp.dot(h, As_ref[...], preferred_element_type=jnp.float32)
    ad_ref[...] = jnp.dot(h, Ad_ref[...], preferred_element_type=jnp.float32)


def _tc1(xp, W1, As, Ad):
    return pl.pallas_call(
        _tc1_body,
        grid=(NP // BN1,),
        in_specs=[
            pl.BlockSpec((BN1, F_IN), lambda i: (i, 0)),
            pl.BlockSpec((F_IN, HEADS * HID), lambda i: (0, 0)),
            pl.BlockSpec((HEADS * HID, HEADS), lambda i: (0, 0)),
            pl.BlockSpec((HEADS * HID, HEADS), lambda i: (0, 0)),
        ],
        out_specs=[
            pl.BlockSpec((BN1, HEADS * HID), lambda i: (i, 0)),
            pl.BlockSpec((BN1, HEADS), lambda i: (i, 0)),
            pl.BlockSpec((BN1, HEADS), lambda i: (i, 0)),
        ],
        out_shape=[
            jax.ShapeDtypeStruct((NP, HEADS * HID), jnp.float32),
            jax.ShapeDtypeStruct((NP, HEADS), jnp.float32),
            jax.ShapeDtypeStruct((NP, HEADS), jnp.float32),
        ],
    )(xp, W1, As, Ad)


def _tc2_body(or_ref, dent_ref, b1_ref, W2_ref, a2m_ref, h2_ref, asd_ref):
    f32 = jnp.float32
    parts = []
    for g in range(G):
        d0 = jnp.broadcast_to(dent_ref[g, :, 0:1], (BN2, HID))
        d1 = jnp.broadcast_to(dent_ref[g, :, 1:2], (BN2, HID))
        dcat = jnp.concatenate([d0, d1], axis=1)
        parts.append(or_ref[g] / (dcat + f32(1e-16)))
    h1 = jnp.concatenate(parts, axis=1) + b1_ref[...]
    h1 = jnp.where(h1 > 0, h1, jnp.exp(jnp.minimum(h1, 0.0)) - f32(1.0))
    h2 = jnp.dot(h1, W2_ref[...], preferred_element_type=f32)
    h2_ref[...] = h2
    asd_ref[...] = jnp.dot(h2, a2m_ref[...], preferred_element_type=f32)


def _tc2(or3, dent, b1r, W2, a2m):
    return pl.pallas_call(
        _tc2_body,
        grid=(NP // BN2,),
        in_specs=[
            pl.BlockSpec((G, BN2, 2 * HID), lambda i: (0, i, 0)),
            pl.BlockSpec((G, BN2, 2), lambda i: (0, i, 0)),
            pl.BlockSpec((1, HEADS * HID), lambda i: (0, 0)),
            pl.BlockSpec((HEADS * HID, 2), lambda i: (0, 0)),
            pl.BlockSpec((2, 2), lambda i: (0, 0)),
        ],
        out_specs=[
            pl.BlockSpec((BN2, 2), lambda i: (i, 0)),
            pl.BlockSpec((BN2, 2), lambda i: (i, 0)),
        ],
        out_shape=[
            jax.ShapeDtypeStruct((NP, 2), jnp.float32),
            jax.ShapeDtypeStruct((NP, 2), jnp.float32),
        ],
    )(or3, dent, b1r, W2, a2m)


def _sc2_body(as2T, ad2T, h2c, src, dst, bm2, b2b, zflat,
              out2,
              o0s, o1s, d2s,
              as2v, ad2v, h0v, h1v, srcS, dstS,
              dstCA, eabA, v0bA, v1bA, dstCB, eabB, v0bB, v1bB,
              bm2v, b2v, s0, s1, s2, ob, sem1):
    c = lax.axis_index("c")
    t = lax.axis_index("s")
    eo = t * EPT
    ro = t * RPT

    @pl.when(c == 0)
    def _():
        pltpu.sync_copy(as2T, as2v)
        pltpu.sync_copy(ad2T, ad2v)
        pltpu.sync_copy(h2c.at[pl.ds(0, NP)], h0v)
        pltpu.sync_copy(h2c.at[pl.ds(NP, NP)], h1v)
        pltpu.sync_copy(bm2, bm2v)
        pltpu.sync_copy(b2b, b2v)
        pltpu.sync_copy(zflat.at[pl.ds(ro, RPT)], o0s.at[pl.ds(ro, RPT)])
        pltpu.sync_copy(zflat.at[pl.ds(ro, RPT)], o1s.at[pl.ds(ro, RPT)])
        pltpu.sync_copy(zflat.at[pl.ds(ro, RPT)], d2s.at[pl.ds(ro, RPT)])
        plsc.subcore_barrier()
        bmv = bm2v[...]

        def super_body(sidx, carry):
            ebase = eo + sidx * (CPS * CH)
            pltpu.sync_copy(src.at[pl.ds(ebase, CPS * CH)], srcS)
            pltpu.sync_copy(dst.at[pl.ds(ebase, CPS * CH)], dstS)
            prev = ()
            for cc in range(CPS):
                dstC, eab, v0b, v1b = ((dstCA, eabA, v0bA, v1bA) if cc % 2 == 0
                                       else (dstCB, eabB, v0bB, v1bB))
                for s in range(CH // L):
                    sv = srcS[pl.ds(cc * CH + s * L, L)]
                    dv = dstS[pl.ds(cc * CH + s * L, L)]
                    dstC[pl.ds(s * L, L)] = dv
                    a = plsc.load_gather(as2v, [sv]) + plsc.load_gather(ad2v, [dv])
                    a = jnp.where(a > 0, a, a * jnp.float32(0.2))
                    ea = jnp.exp(a - bmv)
                    eab[pl.ds(s * L, L)] = ea
                    v0b[pl.ds(s * L, L)] = ea * plsc.load_gather(h0v, [sv])
                    v1b[pl.ds(s * L, L)] = ea * plsc.load_gather(h1v, [sv])
                for d in prev:
                    d.wait()
                prev = (
                    pltpu.async_copy(eab, d2s.at[dstC], sem1, add=True),
                    pltpu.async_copy(v0b, o0s.at[dstC], sem1, add=True),
                    pltpu.async_copy(v1b, o1s.at[dstC], sem1, add=True),
                )
            for d in prev:
                d.wait()
            return carry

        lax.fori_loop(0, NSUP, super_body, 0)
        plsc.subcore_barrier()
        pltpu.sync_copy(o0s.at[pl.ds(ro, RPT)], s0)
        pltpu.sync_copy(o1s.at[pl.ds(ro, RPT)], s1)
        pltpu.sync_copy(d2s.at[pl.ds(ro, RPT)], s2)
        bb0 = b2v[pl.ds(0, L)]
        bb1 = b2v[pl.ds(L, L)]

        def out_body(i, carry):
            sl = pl.ds(i * L, L)
            d = s2[sl] + jnp.float32(1e-16)
            ob[sl] = s0[sl] / d + bb0
            s0[sl] = s1[sl] / d + bb1
            return carry

        lax.fori_loop(0, RPT // L, out_body, 0)
        pltpu.sync_copy(ob, out2.at[pl.ds(ro, RPT)])
        pltpu.sync_copy(s0, out2.at[pl.ds(NP + ro, RPT)])


def _sc2(as2T, ad2T, h2c, src, dst, bm2, b2b, zflat):
    mesh = plsc.VectorSubcoreMesh(core_axis_name="c", subcore_axis_name="s")
    f = pl.kernel(
        _sc2_body,
        out_type=[jax.ShapeDtypeStruct((2 * NP,), jnp.float32)],
        mesh=mesh,
        compiler_params=pltpu.CompilerParams(needs_layout_passes=False),
        scratch_types=[
            pltpu.VMEM_SHARED((NP,), jnp.float32),
            pltpu.VMEM_SHARED((NP,), jnp.float32),
            pltpu.VMEM_SHARED((NP,), jnp.float32),
            pltpu.VMEM((NP,), jnp.float32),
            pltpu.VMEM((NP,), jnp.float32),
            pltpu.VMEM((NP,), jnp.float32),
            pltpu.VMEM((NP,), jnp.float32),
            pltpu.VMEM((CPS * CH,), jnp.int32),
            pltpu.VMEM((CPS * CH,), jnp.int32),
            pltpu.VMEM((CH,), jnp.int32),
            pltpu.VMEM((CH,), jnp.float32),
            pltpu.VMEM((CH,), jnp.float32),
            pltpu.VMEM((CH,), jnp.float32),
            pltpu.VMEM((CH,), jnp.int32),
            pltpu.VMEM((CH,), jnp.float32),
            pltpu.VMEM((CH,), jnp.float32),
            pltpu.VMEM((CH,), jnp.float32),
            pltpu.VMEM((L,), jnp.float32),
            pltpu.VMEM((2 * L,), jnp.float32),
            pltpu.VMEM((RPT,), jnp.float32),
            pltpu.VMEM((RPT,), jnp.float32),
            pltpu.VMEM((RPT,), jnp.float32),
            pltpu.VMEM((RPT,), jnp.float32),
            pltpu.SemaphoreType.DMA,
        ],
    )
    return f(as2T, ad2T, h2c, src, dst, bm2, b2b, zflat)


def _scA_body(asrcT, adstT, src, dst, bmax, zflat,
              eah, den,
              dtot0, dtot1,
              asrc0v, asrc1v, adst0v, adst1v,
              srcS, dstS, dstCA, dstCB, ea0A, ea1A, ea0B, ea1B,
              bmaxv, sem1, sem2):
    c = lax.axis_index("c")
    t = lax.axis_index("s")
    eo = t * EPT
    ro = t * RPT
    pltpu.sync_copy(bmax, bmaxv)

    def group_body(j, carry):
        g = 2 * c + j
        h0 = 2 * g
        pltpu.sync_copy(asrcT.at[pl.ds(h0 * NP, NP)], asrc0v)
        pltpu.sync_copy(asrcT.at[pl.ds((h0 + 1) * NP, NP)], asrc1v)
        pltpu.sync_copy(adstT.at[pl.ds(h0 * NP, NP)], adst0v)
        pltpu.sync_copy(adstT.at[pl.ds((h0 + 1) * NP, NP)], adst1v)
        pltpu.sync_copy(zflat.at[pl.ds(ro, RPT)], dtot0.at[pl.ds(ro, RPT)])
        pltpu.sync_copy(zflat.at[pl.ds(ro, RPT)], dtot1.at[pl.ds(ro, RPT)])
        plsc.subcore_barrier()
        b0 = plsc.load_gather(bmaxv, [jnp.full((L,), 0, jnp.int32) + h0])
        b1 = plsc.load_gather(bmaxv, [jnp.full((L,), 1, jnp.int32) + h0])

        def compute_ea(cc, dstC, ea0, ea1):
            for s in range(CH // L):
                sv = srcS[pl.ds(cc * CH + s * L, L)]
                dv = dstS[pl.ds(cc * CH + s * L, L)]
                dstC[pl.ds(s * L, L)] = dv
                a0 = plsc.load_gather(asrc0v, [sv]) + plsc.load_gather(adst0v, [dv])
                a0 = jnp.where(a0 > 0, a0, a0 * jnp.float32(0.2))
                ea0[pl.ds(s * L, L)] = jnp.exp(a0 - b0)
                a1 = plsc.load_gather(asrc1v, [sv]) + plsc.load_gather(adst1v, [dv])
                a1 = jnp.where(a1 > 0, a1, a1 * jnp.float32(0.2))
                ea1[pl.ds(s * L, L)] = jnp.exp(a1 - b1)

        def super_body(sidx, carry2):
            ebase = eo + sidx * (CPS * CH)
            pltpu.sync_copy(src.at[pl.ds(ebase, CPS * CH)], srcS)
            pltpu.sync_copy(dst.at[pl.ds(ebase, CPS * CH)], dstS)
            prev = ()
            for cc in range(CPS):
                dstC, ea0, ea1 = ((dstCA, ea0A, ea1A) if cc % 2 == 0
                                  else (dstCB, ea0B, ea1B))
                compute_ea(cc, dstC, ea0, ea1)
                for d in prev:
                    d.wait()
                prev = (
                    pltpu.async_copy(ea0, dtot0.at[dstC], sem1, add=True),
                    pltpu.async_copy(ea1, dtot1.at[dstC], sem1, add=True),
                    pltpu.async_copy(
                        ea0, eah.at[pl.ds(h0 * E + ebase + cc * CH, CH)], sem2),
                    pltpu.async_copy(
                        ea1, eah.at[pl.ds((h0 + 1) * E + ebase + cc * CH, CH)], sem2),
                )
            for d in prev:
                d.wait()
            return carry2

        lax.fori_loop(0, NSUP, super_body, 0)
        plsc.subcore_barrier()
        pltpu.sync_copy(dtot0.at[pl.ds(ro, RPT)],
                        den.at[pl.ds(h0 * NP + ro, RPT)])
        pltpu.sync_copy(dtot1.at[pl.ds(ro, RPT)],
                        den.at[pl.ds((h0 + 1) * NP + ro, RPT)])
        plsc.subcore_barrier()
        return carry

    lax.fori_loop(0, 2, group_body, 0)


def _scB_body(hg, src, dst, eah, zrow,
              out1,
              acc,
              srcS, dstS, ea0S, ea1S,
              adjA, rowsA, dstCA, adjA2, rowsA2, dstCA2,
              adjB, rowsB, dstCB, adjB2, rowsB2, dstCB2,
              semA, semSA, semB, semSB):
    c = lax.axis_index("c")
    t = lax.axis_index("s")
    eo = t * EPT
    ro = t * RPT

    def group_body(j, carry):
        g = 2 * c + j
        h0 = 2 * g
        goff = g * NP
        pltpu.sync_copy(zrow, acc.at[pl.ds(ro, RPT)])
        plsc.subcore_barrier()

        def build_adj(cc, adj, dstC):
            for s in range(CH // L):
                adj[pl.ds(s * L, L)] = srcS[pl.ds(cc * CH + s * L, L)] + goff
                dstC[pl.ds(s * L, L)] = dstS[pl.ds(cc * CH + s * L, L)]

        def scale_scatter(cc, rows):
            def scale_body(eb, carry3):
                for u in range(UNROLL):
                    e = eb * UNROLL + u
                    ev = jnp.full((L,), cc * CH, jnp.int32) + e
                    sp0 = plsc.load_gather(ea0S, [ev])
                    sp1 = plsc.load_gather(ea1S, [ev])
                    for r in range(8):
                        sp = sp0 if r < 4 else sp1
                        sl = pl.ds(r * L, L)
                        rows[e, sl] = rows[e, sl] * sp
                return carry3

            lax.fori_loop(0, CH // UNROLL, scale_body, 0)

        sets = ((adjA, rowsA, dstCA, adjA2, rowsA2, dstCA2, semA, semSA),
                (adjB, rowsB, dstCB, adjB2, rowsB2, dstCB2, semB, semSB))

        def stage_unit(u, st):
            a1, r1, d1, a2, r2, d2, sg, ss = st
            build_adj(2 * u, a1, d1)
            build_adj(2 * u + 1, a2, d2)
            pltpu.async_copy(hg.at[a1], r1, sg)
            pltpu.async_copy(hg.at[a2], r2, sg)

        def wait_gathers(st):
            a1, r1, d1, a2, r2, d2, sg, ss = st
            pltpu.make_async_copy(hg.at[a1], r1, sg).wait()
            pltpu.make_async_copy(hg.at[a2], r2, sg).wait()

        def drain_scatter(st):
            a1, r1, d1, a2, r2, d2, sg, ss = st
            pltpu.make_async_copy(r1, acc.at[d1], ss).wait()
            pltpu.make_async_copy(r2, acc.at[d2], ss).wait()

        def scale_fire(u, st):
            a1, r1, d1, a2, r2, d2, sg, ss = st
            scale_scatter(2 * u, r1)
            scale_scatter(2 * u + 1, r2)
            pltpu.async_copy(r1, acc.at[d1], ss, add=True)
            pltpu.async_copy(r2, acc.at[d2], ss, add=True)

        def super_body(sidx, carry2):
            ebase = eo + sidx * (CPS * CH)
            pltpu.sync_copy(src.at[pl.ds(ebase, CPS * CH)], srcS)
            pltpu.sync_copy(dst.at[pl.ds(ebase, CPS * CH)], dstS)
            pltpu.sync_copy(eah.at[pl.ds(h0 * E + ebase, CPS * CH)], ea0S)
            pltpu.sync_copy(eah.at[pl.ds((h0 + 1) * E + ebase, CPS * CH)], ea1S)
            stage_unit(0, sets[0])
            for u in range(CPS // 2):
                p = u % 2
                wait_gathers(sets[p])
                if u >= 1:
                    drain_scatter(sets[1 - p])
                if u < CPS // 2 - 1:
                    stage_unit(u + 1, sets[1 - p])
                scale_fire(u, sets[p])
            drain_scatter(sets[0])
            return carry2

        lax.fori_loop(0, NSUP, super_body, 0)
        plsc.subcore_barrier()
        pltpu.sync_copy(acc.at[pl.ds(ro, RPT)],
                        out1.at[pl.ds(goff + ro, RPT)])
        plsc.subcore_barrier()
        return carry

    lax.fori_loop(0, 2, group_body, 0)


@jax.jit
def _sc1(hg, asrcT, adstT, src, dst, bmax, zrow, zflat):
    mesh = plsc.VectorSubcoreMesh(core_axis_name="c", subcore_axis_name="s")
    fA = pl.kernel(
        _scA_body,
        out_type=[
            jax.ShapeDtypeStruct((HEADS * E,), jnp.float32),
            jax.ShapeDtypeStruct((HEADS * NP,), jnp.float32),
        ],
        mesh=mesh,
        compiler_params=pltpu.CompilerParams(needs_layout_passes=False),
        scratch_types=[
            pltpu.VMEM_SHARED((NP,), jnp.float32),
            pltpu.VMEM_SHARED((NP,), jnp.float32),
            pltpu.VMEM((NP,), jnp.float32),
            pltpu.VMEM((NP,), jnp.float32),
            pltpu.VMEM((NP,), jnp.float32),
            pltpu.VMEM((NP,), jnp.float32),
            pltpu.VMEM((CPS * CH,), jnp.int32),
            pltpu.VMEM((CPS * CH,), jnp.int32),
            pltpu.VMEM((CH,), jnp.int32),
            pltpu.VMEM((CH,), jnp.int32),
            pltpu.VMEM((CH,), jnp.float32),
            pltpu.VMEM((CH,), jnp.float32),
            pltpu.VMEM((CH,), jnp.float32),
            pltpu.VMEM((CH,), jnp.float32),
            pltpu.VMEM((L,), jnp.float32),
            pltpu.SemaphoreType.DMA,
            pltpu.SemaphoreType.DMA,
        ],
    )
    eah, den = fA(asrcT, adstT, src, dst, bmax, zflat)
    fB = pl.kernel(
        _scB_body,
        out_type=[
            jax.ShapeDtypeStruct((G * NP, 128), jnp.float32),
        ],
        mesh=mesh,
        compiler_params=pltpu.CompilerParams(needs_layout_passes=False),
        scratch_types=[
            pltpu.VMEM_SHARED((NP, 128), jnp.float32),
            pltpu.VMEM((CPS * CH,), jnp.int32),
            pltpu.VMEM((CPS * CH,), jnp.int32),
            pltpu.VMEM((CPS * CH,), jnp.float32),
            pltpu.VMEM((CPS * CH,), jnp.float32),
            pltpu.VMEM((CH,), jnp.int32),
            pltpu.VMEM((CH, 128), jnp.float32),
            pltpu.VMEM((CH,), jnp.int32),
            pltpu.VMEM((CH,), jnp.int32),
            pltpu.VMEM((CH, 128), jnp.float32),
            pltpu.VMEM((CH,), jnp.int32),
            pltpu.VMEM((CH,), jnp.int32),
            pltpu.VMEM((CH, 128), jnp.float32),
            pltpu.VMEM((CH,), jnp.int32),
            pltpu.VMEM((CH,), jnp.int32),
            pltpu.VMEM((CH, 128), jnp.float32),
            pltpu.VMEM((CH,), jnp.int32),
            pltpu.SemaphoreType.DMA,
            pltpu.SemaphoreType.DMA,
            pltpu.SemaphoreType.DMA,
            pltpu.SemaphoreType.DMA,
        ],
    )
    (out1f,) = fB(hg, src, dst, eah, zrow)
    return out1f, den


def kernel(x, edge_index, W1, att_src1, att_dst1, b1, W2, att_src2, att_dst2, b2):
    f32 = jnp.float32
    src = edge_index[0]
    dst = edge_index[1]
    xp = jnp.pad(x, ((0, NP - N), (0, 0)))

    As = jnp.zeros((HEADS * HID, HEADS), f32)
    Ad = jnp.zeros((HEADS * HID, HEADS), f32)
    for hh in range(HEADS):
        As = As.at[hh * HID:(hh + 1) * HID, hh].set(att_src1[hh])
        Ad = Ad.at[hh * HID:(hh + 1) * HID, hh].set(att_dst1[hh])

    h, asrc, adst = _tc1(xp, W1, As, Ad)

    gm = asrc.max(0) + adst.max(0)
    bmax = jnp.where(gm > 0, gm, gm * f32(0.2))
    bmax16 = jnp.pad(bmax, (0, 16 - HEADS))
    hg = h.reshape(NP, G, 128).transpose(1, 0, 2).reshape(G * NP, 128)
    asrcT = asrc.T.reshape(-1)
    adstT = adst.T.reshape(-1)
    zrow = jnp.zeros((RPT, 128), f32)
    zflat = jnp.zeros((NP,), f32)

    out1f, denf = _sc1(hg, asrcT, adstT, src, dst, bmax16, zrow, zflat)

    or3 = out1f.reshape(G, NP, 128)
    dent = denf.reshape(G, 2, NP).transpose(0, 2, 1)
    a2m = jnp.stack([att_src2[0], att_dst2[0]], axis=1)
    h2, asd = _tc2(or3, dent, b1.reshape(1, HEADS * HID), W2, a2m)

    as2 = asd[:, 0]
    ad2 = asd[:, 1]
    gm2 = as2.max() + ad2.max()
    bm2 = jnp.where(gm2 > 0, gm2, gm2 * f32(0.2))
    bm2v = jnp.broadcast_to(bm2[None], (L,))
    b2b = jnp.broadcast_to(b2[:, None], (2, L)).reshape(-1)
    h2c = h2.T.reshape(-1)

    (out2,) = _sc2(as2, ad2, h2c, src, dst, bm2v, b2b, zflat)
    return out2.reshape(2, NP).T[:N]
